# Initial kernel scaffold; baseline (speedup 1.0000x reference)
import jax
import jax.numpy as jnp
from jax.experimental import pallas as pl


def kernel(x, edge_index, Wl1_artist, bl1_artist, Wr1_artist, Wlo_artist, blo_artist, Wro_artist, Wl1_style, bl1_style, Wr1_style, Wlo_style, blo_style, Wro_style, Wl1_genre, bl1_genre, Wr1_genre, Wlo_genre, blo_genre, Wro_genre):
    raise NotImplementedError("write your pallas kernel here")


# speedup vs baseline: 3.1404x; 3.1404x over previous
import functools

import jax
import jax.numpy as jnp
from jax import lax
from jax.experimental import pallas as pl
from jax.experimental.pallas import tpu as pltpu
from jax.experimental.pallas import tpu_sc as plsc

N = 10000
E = 320000
D = 128
H = 128

NC = 2
NS = 16
CE = 80

C1 = 144
C2 = 96

ZR = 624
ZTAIL_BASE = ZR * 15
ZTAIL = N - ZTAIL_BASE - ZR


def _zero_and_barrier(zeros_hbm, acc, s):
    pltpu.sync_copy(zeros_hbm.at[pl.ds(s * ZR, ZR)], acc.at[pl.ds(s * ZR, ZR)])

    @pl.when(s == NS - 1)
    def _():
        pltpu.sync_copy(zeros_hbm.at[pl.ds(ZTAIL_BASE + ZR, ZTAIL)],
                        acc.at[pl.ds(ZTAIL_BASE + ZR, ZTAIL)])

    plsc.subcore_barrier()


def _writeback(acc, out_hbm, c, s):
    pltpu.sync_copy(acc.at[pl.ds(s * ZR, ZR)], out_hbm.at[c, pl.ds(s * ZR, ZR)])

    @pl.when(s == NS - 1)
    def _():
        pltpu.sync_copy(acc.at[pl.ds(ZTAIL_BASE + ZR, ZTAIL)],
                        out_hbm.at[c, pl.ds(ZTAIL_BASE + ZR, ZTAIL)])


def _sc_pass1(xaug, src, dst, zeros1):
    n_chunks = E // (NC * NS) // CE

    @functools.partial(
        pl.kernel,
        out_type=jax.ShapeDtypeStruct((NC, N, C1), jnp.float32),
        mesh=plsc.VectorSubcoreMesh(core_axis_name="c", subcore_axis_name="s"),
        compiler_params=pltpu.CompilerParams(use_tc_tiling_on_sc=False),
        scratch_types=[
            pltpu.VMEM((CE,), jnp.int32),
            pltpu.VMEM((CE,), jnp.int32),
            pltpu.VMEM((CE, C1), jnp.float32),
            pltpu.VMEM_SHARED((N, C1), jnp.float32),
            pltpu.SemaphoreType.DMA,
        ],
    )
    def k(xaug_h, src_h, dst_h, zeros_h, out_h, srcv, dstv, rows, acc, sem):
        c = lax.axis_index("c")
        s = lax.axis_index("s")
        _zero_and_barrier(zeros_h, acc, s)
        tid = c * NS + s
        edge_base = tid * (E // (NC * NS))

        def chunk(i, _):
            base = edge_base + i * CE
            pltpu.sync_copy(src_h.at[pl.ds(base, CE)], srcv)
            pltpu.sync_copy(dst_h.at[pl.ds(base, CE)], dstv)
            pltpu.async_copy(xaug_h.at[srcv], rows, sem).wait()
            pltpu.sync_copy(rows, acc.at[dstv], add=True)
            return _

        lax.fori_loop(0, n_chunks, chunk, 0)
        plsc.subcore_barrier()
        _writeback(acc, out_h, c, s)

    return k(xaug, src, dst, zeros1)


def _sc_pass2(tbl_flat, src4, dst, zeros2):
    n_chunks = E // NS // CE

    @functools.partial(
        pl.kernel,
        out_type=jax.ShapeDtypeStruct((2 * NC, N, C2), jnp.float32),
        mesh=plsc.VectorSubcoreMesh(core_axis_name="c", subcore_axis_name="s"),
        compiler_params=pltpu.CompilerParams(use_tc_tiling_on_sc=False),
        scratch_types=[
            pltpu.VMEM((CE,), jnp.int32),
            pltpu.VMEM((CE,), jnp.int32),
            pltpu.VMEM((CE, C2), jnp.float32),
            pltpu.VMEM_SHARED((N, C2), jnp.float32),
            pltpu.SemaphoreType.DMA,
        ],
    )
    def k(tbl_h, src4_h, dst_h, zeros_h, out_h, srcv, dstv, rows, acc, sem):
        c = lax.axis_index("c")
        s = lax.axis_index("s")
        edge_base = s * (E // NS)
        for p in range(2):
            q = p * NC + c
            _zero_and_barrier(zeros_h, acc, s)

            def chunk(i, _):
                base = edge_base + i * CE
                pltpu.sync_copy(src4_h.at[q, pl.ds(base, CE)], srcv)
                pltpu.sync_copy(dst_h.at[pl.ds(base, CE)], dstv)
                pltpu.async_copy(tbl_h.at[srcv], rows, sem).wait()
                pltpu.sync_copy(rows, acc.at[dstv], add=True)
                return _

            lax.fori_loop(0, n_chunks, chunk, 0)
            plsc.subcore_barrier()
            _writeback(acc, out_h, q, s)

    return k(tbl_flat, src4, dst, zeros2)


RB = 2000


def _stage_b_body(s1p, xb, wla, bla, wra, wls, bls, wrs, wlg, blg, wrg,
                  ha, hs, hg, emb2, rcnt):
    s1 = s1p[0] + s1p[1]
    cnt = s1[:, D:D + 1]
    rc = 1.0 / jnp.maximum(cnt, 1.0)
    agg = s1[:, :D] * rc
    x = xb[...]

    def head(wl, bl, wr):
        return (jnp.dot(agg, wl[...], preferred_element_type=jnp.float32)
                + bl[...]
                + jnp.dot(x, wr[...], preferred_element_type=jnp.float32))

    h_a = head(wla, bla, wra)
    h_s = head(wls, bls, wrs)
    h_g = head(wlg, blg, wrg)
    ha[...] = h_a
    hs[...] = h_s
    hg[...] = h_g
    full_e = jnp.maximum(jnp.concatenate([h_a, h_s, h_g], axis=1), 0.0)
    for q in range(4):
        emb2[q] = full_e[:, q * C2:(q + 1) * C2]
    rcnt[...] = jnp.broadcast_to(rc, (RB, 8))


def _tc_stage_b(s1p, x, wla, bla, wra, wls, bls, wrs, wlg, blg, wrg):
    grid = (N // RB,)
    full = lambda shape: pl.BlockSpec(shape, lambda i: (0,) * len(shape))
    row = lambda w: pl.BlockSpec((RB, w), lambda i: (i, 0))
    return pl.pallas_call(
        _stage_b_body,
        grid=grid,
        in_specs=[
            pl.BlockSpec((NC, RB, C1), lambda i: (0, i, 0)),
            row(D),
            full((D, H)), full((1, H)), full((D, H)),
            full((D, H)), full((1, H)), full((D, H)),
            full((D, H)), full((1, H)), full((D, H)),
        ],
        out_specs=[
            row(H), row(H), row(H),
            pl.BlockSpec((2 * NC, RB, C2), lambda i: (0, i, 0)),
            row(8),
        ],
        out_shape=[
            jax.ShapeDtypeStruct((N, H), jnp.float32),
            jax.ShapeDtypeStruct((N, H), jnp.float32),
            jax.ShapeDtypeStruct((N, H), jnp.float32),
            jax.ShapeDtypeStruct((2 * NC, N, C2), jnp.float32),
            jax.ShapeDtypeStruct((N, 8), jnp.float32),
        ],
    )(s1p, x, wla, bla, wra, wls, bls, wrs, wlg, blg, wrg)


def _log_softmax(xo):
    m = jnp.max(xo, axis=1, keepdims=True)
    e = jnp.exp(xo - m)
    return xo - m - jnp.log(jnp.sum(e, axis=1, keepdims=True))


def _stage_d_body(s2, rcnt, emb2, wa, ba, ra, ws, bs, rs, wg, bg, rg,
                  ya, ys, yg):
    rc = rcnt[:, :1]
    s2_full = jnp.concatenate([s2[q] for q in range(4)], axis=1)
    e_full = jnp.concatenate([emb2[q] for q in range(4)], axis=1)
    agg_a = s2_full[:, :H] * rc
    agg_s = s2_full[:, H:2 * H] * rc
    agg_g = s2_full[:, 2 * H:] * rc
    e_a = e_full[:, :H]
    e_s = e_full[:, H:2 * H]
    e_g = e_full[:, 2 * H:]

    def head(agg, emb, wl, bl, wr):
        return (jnp.dot(agg, wl[...], preferred_element_type=jnp.float32)
                + bl[...]
                + jnp.dot(emb, wr[...], preferred_element_type=jnp.float32))

    ya[...] = _log_softmax(head(agg_a, e_a, wa, ba, ra))
    ys[...] = _log_softmax(head(agg_s, e_s, ws, bs, rs))
    yg[...] = _log_softmax(head(agg_g, e_g, wg, bg, rg))


def _tc_stage_d(s2, rcnt, emb2, wa, ba, ra, ws, bs, rs, wg, bg, rg):
    grid = (N // RB,)
    full = lambda shape: pl.BlockSpec(shape, lambda i: (0,) * len(shape))
    row = lambda w: pl.BlockSpec((RB, w), lambda i: (i, 0))
    plane = pl.BlockSpec((2 * NC, RB, C2), lambda i: (0, i, 0))
    oa, os_, og = wa.shape[1], ws.shape[1], wg.shape[1]
    return pl.pallas_call(
        _stage_d_body,
        grid=grid,
        in_specs=[
            plane, row(8), plane,
            full((H, oa)), full((1, oa)), full((H, oa)),
            full((H, os_)), full((1, os_)), full((H, os_)),
            full((H, og)), full((1, og)), full((H, og)),
        ],
        out_specs=[row(oa), row(os_), row(og)],
        out_shape=[
            jax.ShapeDtypeStruct((N, oa), jnp.float32),
            jax.ShapeDtypeStruct((N, os_), jnp.float32),
            jax.ShapeDtypeStruct((N, og), jnp.float32),
        ],
    )(s2, rcnt, emb2, wa, ba, ra, ws, bs, rs, wg, bg, rg)


def _pad_out_head(w, b, wr, padded):
    oc = w.shape[1]
    wp = jnp.pad(w, ((0, 0), (0, padded - oc)))
    rp = jnp.pad(wr, ((0, 0), (0, padded - oc)))
    bp = jnp.pad(b.reshape(1, -1), ((0, 0), (0, padded - oc)),
                 constant_values=-1e30)
    return wp, bp, rp


def kernel(x, edge_index,
           Wl1_artist, bl1_artist, Wr1_artist, Wlo_artist, blo_artist, Wro_artist,
           Wl1_style, bl1_style, Wr1_style, Wlo_style, blo_style, Wro_style,
           Wl1_genre, bl1_genre, Wr1_genre, Wlo_genre, blo_genre, Wro_genre):
    src = edge_index[0].astype(jnp.int32)
    dst = edge_index[1].astype(jnp.int32)

    xaug = jnp.concatenate(
        [x, jnp.ones((N, 1), jnp.float32), jnp.zeros((N, C1 - D - 1), jnp.float32)],
        axis=1)
    zeros1 = jnp.zeros((N, C1), jnp.float32)
    zeros2 = jnp.zeros((N, C2), jnp.float32)
    src4 = src[None, :] + (jnp.arange(4, dtype=jnp.int32) * N)[:, None]

    s1p = _sc_pass1(xaug, src, dst, zeros1)

    ha, hs, hg, emb2, rcnt = _tc_stage_b(
        s1p, x,
        Wl1_artist, bl1_artist.reshape(1, -1), Wr1_artist,
        Wl1_style, bl1_style.reshape(1, -1), Wr1_style,
        Wl1_genre, bl1_genre.reshape(1, -1), Wr1_genre)

    s2 = _sc_pass2(emb2.reshape(4 * N, C2), src4, dst, zeros2)

    wa, ba, ra = _pad_out_head(Wlo_artist, blo_artist, Wro_artist, 256)
    ws, bs, rs = _pad_out_head(Wlo_style, blo_style, Wro_style, 128)
    wg, bg, rg = _pad_out_head(Wlo_genre, blo_genre, Wro_genre, 128)

    ya, ys, yg = _tc_stage_d(s2, rcnt, emb2, wa, ba, ra, ws, bs, rs, wg, bg, rg)

    return (ha, ya[:, :Wlo_artist.shape[1]],
            hs, ys[:, :Wlo_style.shape[1]],
            hg, yg[:, :Wlo_genre.shape[1]])


# speedup vs baseline: 5.5379x; 1.7634x over previous
import functools

import jax
import jax.numpy as jnp
from jax import lax
from jax.experimental import pallas as pl
from jax.experimental.pallas import tpu as pltpu
from jax.experimental.pallas import tpu_sc as plsc

N = 10000
E = 320000
D = 128
H = 128

NC = 2
NS = 16
CE = 80

C1 = 144
C2 = 96

ZR = 624
ZTAIL_BASE = ZR * 15
ZTAIL = N - ZTAIL_BASE - ZR


def _zero_and_barrier(zeros_hbm, acc, s):
    pltpu.sync_copy(zeros_hbm.at[pl.ds(s * ZR, ZR)], acc.at[pl.ds(s * ZR, ZR)])

    @pl.when(s == NS - 1)
    def _():
        pltpu.sync_copy(zeros_hbm.at[pl.ds(ZTAIL_BASE + ZR, ZTAIL)],
                        acc.at[pl.ds(ZTAIL_BASE + ZR, ZTAIL)])

    plsc.subcore_barrier()


def _writeback(acc, out_hbm, c, s):
    pltpu.sync_copy(acc.at[pl.ds(s * ZR, ZR)], out_hbm.at[c, pl.ds(s * ZR, ZR)])

    @pl.when(s == NS - 1)
    def _():
        pltpu.sync_copy(acc.at[pl.ds(ZTAIL_BASE + ZR, ZTAIL)],
                        out_hbm.at[c, pl.ds(ZTAIL_BASE + ZR, ZTAIL)])


def _emit_sweep(n, nb, ce, src_ix, dst_ix, tbl, acc, srcb, dstb, rows, sems):
    nbb = 2 * nb
    gsems, ssems = sems[:nb], sems[nb:]

    def issue_gather(j, bb):
        b, rb = bb % nb, bb % nbb
        pltpu.sync_copy(src_ix(j), srcb.at[b])
        pltpu.sync_copy(dst_ix(j), dstb.at[rb])
        pltpu.async_copy(tbl.at[srcb.at[b]], rows.at[rb], gsems[b])

    def wait_scatter(bb):
        b, rb = bb % nb, bb % nbb
        pltpu.make_async_copy(rows.at[rb], acc.at[dstb.at[rb]], ssems[b]).wait()

    def process(j, bb, wait_prev, prefetch):
        b, rb = bb % nb, bb % nbb
        if wait_prev:
            wait_scatter(bb + nb)
        pltpu.make_async_copy(tbl.at[srcb.at[b]], rows.at[rb], gsems[b]).wait()
        pltpu.async_copy(rows.at[rb], acc.at[dstb.at[rb]], ssems[b], add=True)
        if prefetch:
            issue_gather(j + nb, bb + nb)

    ngrp = n // nbb
    for bb in range(nb):
        issue_gather(bb, bb)
    for bb in range(nbb):
        process(bb, bb, bb >= nb, bb + nb < n)

    def grp(g, _):
        for bb in range(nbb):
            process(g * nbb + bb, bb, True, True)
        return _

    lax.fori_loop(1, ngrp - 1, grp, 0)
    for bb in range(nbb):
        j = (ngrp - 1) * nbb + bb
        process(j, bb, True, j + nb < n)
    for t in range(n - ngrp * nbb):
        j = ngrp * nbb + t
        process(j, t, True, j + nb < n)
    for j in range(n - nb, n):
        wait_scatter(j % nbb)


def _sweep_scratch(nb, ce, width):
    return [
        pltpu.VMEM((nb, ce), jnp.int32),
        pltpu.VMEM((2 * nb, ce), jnp.int32),
        pltpu.VMEM((2 * nb, ce, width), jnp.float32),
    ] + [pltpu.SemaphoreType.DMA] * (2 * nb)


NB1, CE1 = 3, 40
NB2, CE2 = 4, 80


def _sc_pass1(xaug, src, dst, zeros1):
    n_chunks = E // (NC * NS) // CE1

    @functools.partial(
        pl.kernel,
        out_type=jax.ShapeDtypeStruct((NC, N, C1), jnp.float32),
        mesh=plsc.VectorSubcoreMesh(core_axis_name="c", subcore_axis_name="s"),
        compiler_params=pltpu.CompilerParams(use_tc_tiling_on_sc=False),
        scratch_types=_sweep_scratch(NB1, CE1, C1) + [pltpu.VMEM_SHARED((N, C1), jnp.float32)],
    )
    def k(xaug_h, src_h, dst_h, zeros_h, out_h, srcb, dstb, rows, *rest):
        sems, acc = list(rest[:2 * NB1]), rest[2 * NB1]
        c = lax.axis_index("c")
        s = lax.axis_index("s")
        _zero_and_barrier(zeros_h, acc, s)
        tid = c * NS + s
        edge_base = tid * (E // (NC * NS))
        _emit_sweep(
            n_chunks, NB1, CE1,
            lambda j: src_h.at[pl.ds(edge_base + j * CE1, CE1)],
            lambda j: dst_h.at[pl.ds(edge_base + j * CE1, CE1)],
            xaug_h, acc, srcb, dstb, rows, sems)
        plsc.subcore_barrier()
        _writeback(acc, out_h, c, s)

    return k(xaug, src, dst, zeros1)


def _sc_pass2(tbl_flat, src4, dst, zeros2):
    n_chunks = E // NS // CE2

    @functools.partial(
        pl.kernel,
        out_type=jax.ShapeDtypeStruct((2 * NC, N, C2), jnp.float32),
        mesh=plsc.VectorSubcoreMesh(core_axis_name="c", subcore_axis_name="s"),
        compiler_params=pltpu.CompilerParams(use_tc_tiling_on_sc=False),
        scratch_types=_sweep_scratch(NB2, CE2, C2) + [pltpu.VMEM_SHARED((N, C2), jnp.float32)],
    )
    def k(tbl_h, src4_h, dst_h, zeros_h, out_h, srcb, dstb, rows, *rest):
        sems, acc = list(rest[:2 * NB2]), rest[2 * NB2]
        c = lax.axis_index("c")
        s = lax.axis_index("s")
        edge_base = s * (E // NS)
        for p in range(2):
            q = p * NC + c
            _zero_and_barrier(zeros_h, acc, s)
            _emit_sweep(
                n_chunks, NB2, CE2,
                lambda j: src4_h.at[q, pl.ds(edge_base + j * CE2, CE2)],
                lambda j: dst_h.at[pl.ds(edge_base + j * CE2, CE2)],
                tbl_h, acc, srcb, dstb, rows, sems)
            plsc.subcore_barrier()
            _writeback(acc, out_h, q, s)

    return k(tbl_flat, src4, dst, zeros2)


RB = 2000


def _stage_b_body(s1p, xb, wla, bla, wra, wls, bls, wrs, wlg, blg, wrg,
                  ha, hs, hg, emb2, rcnt):
    s1 = s1p[0] + s1p[1]
    cnt = s1[:, D:D + 1]
    rc = 1.0 / jnp.maximum(cnt, 1.0)
    agg = s1[:, :D] * rc
    x = xb[...]

    def head(wl, bl, wr):
        return (jnp.dot(agg, wl[...], preferred_element_type=jnp.float32)
                + bl[...]
                + jnp.dot(x, wr[...], preferred_element_type=jnp.float32))

    h_a = head(wla, bla, wra)
    h_s = head(wls, bls, wrs)
    h_g = head(wlg, blg, wrg)
    ha[...] = h_a
    hs[...] = h_s
    hg[...] = h_g
    full_e = jnp.maximum(jnp.concatenate([h_a, h_s, h_g], axis=1), 0.0)
    for q in range(4):
        emb2[q] = full_e[:, q * C2:(q + 1) * C2]
    rcnt[...] = jnp.broadcast_to(rc, (RB, 8))


def _tc_stage_b(s1p, x, wla, bla, wra, wls, bls, wrs, wlg, blg, wrg):
    grid = (N // RB,)
    full = lambda shape: pl.BlockSpec(shape, lambda i: (0,) * len(shape))
    row = lambda w: pl.BlockSpec((RB, w), lambda i: (i, 0))
    return pl.pallas_call(
        _stage_b_body,
        grid=grid,
        in_specs=[
            pl.BlockSpec((NC, RB, C1), lambda i: (0, i, 0)),
            row(D),
            full((D, H)), full((1, H)), full((D, H)),
            full((D, H)), full((1, H)), full((D, H)),
            full((D, H)), full((1, H)), full((D, H)),
        ],
        out_specs=[
            row(H), row(H), row(H),
            pl.BlockSpec((2 * NC, RB, C2), lambda i: (0, i, 0)),
            row(8),
        ],
        out_shape=[
            jax.ShapeDtypeStruct((N, H), jnp.float32),
            jax.ShapeDtypeStruct((N, H), jnp.float32),
            jax.ShapeDtypeStruct((N, H), jnp.float32),
            jax.ShapeDtypeStruct((2 * NC, N, C2), jnp.float32),
            jax.ShapeDtypeStruct((N, 8), jnp.float32),
        ],
    )(s1p, x, wla, bla, wra, wls, bls, wrs, wlg, blg, wrg)


def _log_softmax(xo):
    m = jnp.max(xo, axis=1, keepdims=True)
    e = jnp.exp(xo - m)
    return xo - m - jnp.log(jnp.sum(e, axis=1, keepdims=True))


def _stage_d_body(s2, rcnt, emb2, wa, ba, ra, ws, bs, rs, wg, bg, rg,
                  ya, ys, yg):
    rc = rcnt[:, :1]
    s2_full = jnp.concatenate([s2[q] for q in range(4)], axis=1)
    e_full = jnp.concatenate([emb2[q] for q in range(4)], axis=1)
    agg_a = s2_full[:, :H] * rc
    agg_s = s2_full[:, H:2 * H] * rc
    agg_g = s2_full[:, 2 * H:] * rc
    e_a = e_full[:, :H]
    e_s = e_full[:, H:2 * H]
    e_g = e_full[:, 2 * H:]

    def head(agg, emb, wl, bl, wr):
        return (jnp.dot(agg, wl[...], preferred_element_type=jnp.float32)
                + bl[...]
                + jnp.dot(emb, wr[...], preferred_element_type=jnp.float32))

    ya[...] = _log_softmax(head(agg_a, e_a, wa, ba, ra))
    ys[...] = _log_softmax(head(agg_s, e_s, ws, bs, rs))
    yg[...] = _log_softmax(head(agg_g, e_g, wg, bg, rg))


def _tc_stage_d(s2, rcnt, emb2, wa, ba, ra, ws, bs, rs, wg, bg, rg):
    grid = (N // RB,)
    full = lambda shape: pl.BlockSpec(shape, lambda i: (0,) * len(shape))
    row = lambda w: pl.BlockSpec((RB, w), lambda i: (i, 0))
    plane = pl.BlockSpec((2 * NC, RB, C2), lambda i: (0, i, 0))
    oa, os_, og = wa.shape[1], ws.shape[1], wg.shape[1]
    return pl.pallas_call(
        _stage_d_body,
        grid=grid,
        in_specs=[
            plane, row(8), plane,
            full((H, oa)), full((1, oa)), full((H, oa)),
            full((H, os_)), full((1, os_)), full((H, os_)),
            full((H, og)), full((1, og)), full((H, og)),
        ],
        out_specs=[row(oa), row(os_), row(og)],
        out_shape=[
            jax.ShapeDtypeStruct((N, oa), jnp.float32),
            jax.ShapeDtypeStruct((N, os_), jnp.float32),
            jax.ShapeDtypeStruct((N, og), jnp.float32),
        ],
    )(s2, rcnt, emb2, wa, ba, ra, ws, bs, rs, wg, bg, rg)


def _pad_out_head(w, b, wr, padded):
    oc = w.shape[1]
    wp = jnp.pad(w, ((0, 0), (0, padded - oc)))
    rp = jnp.pad(wr, ((0, 0), (0, padded - oc)))
    bp = jnp.pad(b.reshape(1, -1), ((0, 0), (0, padded - oc)),
                 constant_values=-1e30)
    return wp, bp, rp


def kernel(x, edge_index,
           Wl1_artist, bl1_artist, Wr1_artist, Wlo_artist, blo_artist, Wro_artist,
           Wl1_style, bl1_style, Wr1_style, Wlo_style, blo_style, Wro_style,
           Wl1_genre, bl1_genre, Wr1_genre, Wlo_genre, blo_genre, Wro_genre):
    src = edge_index[0].astype(jnp.int32)
    dst = edge_index[1].astype(jnp.int32)

    xaug = jnp.concatenate(
        [x, jnp.ones((N, 1), jnp.float32), jnp.zeros((N, C1 - D - 1), jnp.float32)],
        axis=1)
    zeros1 = jnp.zeros((N, C1), jnp.float32)
    zeros2 = jnp.zeros((N, C2), jnp.float32)
    src4 = src[None, :] + (jnp.arange(4, dtype=jnp.int32) * N)[:, None]

    s1p = _sc_pass1(xaug, src, dst, zeros1)

    ha, hs, hg, emb2, rcnt = _tc_stage_b(
        s1p, x,
        Wl1_artist, bl1_artist.reshape(1, -1), Wr1_artist,
        Wl1_style, bl1_style.reshape(1, -1), Wr1_style,
        Wl1_genre, bl1_genre.reshape(1, -1), Wr1_genre)

    s2 = _sc_pass2(emb2.reshape(4 * N, C2), src4, dst, zeros2)

    wa, ba, ra = _pad_out_head(Wlo_artist, blo_artist, Wro_artist, 256)
    ws, bs, rs = _pad_out_head(Wlo_style, blo_style, Wro_style, 128)
    wg, bg, rg = _pad_out_head(Wlo_genre, blo_genre, Wro_genre, 128)

    ya, ys, yg = _tc_stage_d(s2, rcnt, emb2, wa, ba, ra, ws, bs, rs, wg, bg, rg)

    return (ha, ya[:, :Wlo_artist.shape[1]],
            hs, ys[:, :Wlo_style.shape[1]],
            hg, yg[:, :Wlo_genre.shape[1]])


# speedup vs baseline: 8.3258x; 1.5034x over previous
import functools

import jax
import jax.numpy as jnp
from jax import lax
from jax.experimental import pallas as pl
from jax.experimental.pallas import tpu as pltpu
from jax.experimental.pallas import tpu_sc as plsc

N = 10000
E = 320000
D = 128
H = 128

NC = 2
NS = 16
CE = 80

C1 = 144
C2 = 96

ZR = 624
ZTAIL_BASE = ZR * 15
ZTAIL = N - ZTAIL_BASE - ZR


def _zero_and_barrier(zeros_hbm, acc, s):
    pltpu.sync_copy(zeros_hbm.at[pl.ds(s * ZR, ZR)], acc.at[pl.ds(s * ZR, ZR)])

    @pl.when(s == NS - 1)
    def _():
        pltpu.sync_copy(zeros_hbm.at[pl.ds(ZTAIL_BASE + ZR, ZTAIL)],
                        acc.at[pl.ds(ZTAIL_BASE + ZR, ZTAIL)])

    plsc.subcore_barrier()


def _writeback(acc, out_hbm, c, s):
    pltpu.sync_copy(acc.at[pl.ds(s * ZR, ZR)], out_hbm.at[c, pl.ds(s * ZR, ZR)])

    @pl.when(s == NS - 1)
    def _():
        pltpu.sync_copy(acc.at[pl.ds(ZTAIL_BASE + ZR, ZTAIL)],
                        out_hbm.at[c, pl.ds(ZTAIL_BASE + ZR, ZTAIL)])


def _emit_sweep(n, nb, ce, src_ix, dst_ix, tbl, acc, srcb, dstb, rows, sems):
    nbb = 2 * nb
    dd = 2 * nbb
    gsems = sems[:nb]
    ssems = sems[nb:2 * nb]
    isems = sems[2 * nb:2 * nb + nbb]
    dsems = sems[2 * nb + nbb:]

    def issue_idx(j, bb):
        sb, db = bb % nbb, bb % dd
        pltpu.async_copy(src_ix(j), srcb.at[sb], isems[sb])
        pltpu.async_copy(dst_ix(j), dstb.at[db], dsems[sb])

    def issue_gather(j, bb):
        b, sb, rb = bb % nb, bb % nbb, bb % nbb
        pltpu.make_async_copy(src_ix(j), srcb.at[sb], isems[sb]).wait()
        pltpu.async_copy(tbl.at[srcb.at[sb]], rows.at[rb], gsems[b])

    def wait_scatter(bb):
        b, rb = bb % nb, bb % nbb
        pltpu.make_async_copy(rows.at[rb], acc.at[dstb.at[0]], ssems[b]).wait()

    def process(j, bb, wait_prev, pf_idx, pf_gather):
        b, sb, rb, db = bb % nb, bb % nbb, bb % nbb, bb % dd
        if wait_prev:
            wait_scatter(bb + nb)
        pltpu.make_async_copy(tbl.at[srcb.at[sb]], rows.at[rb], gsems[b]).wait()
        pltpu.make_async_copy(dst_ix(j), dstb.at[db], dsems[sb]).wait()
        pltpu.async_copy(rows.at[rb], acc.at[dstb.at[db]], ssems[b], add=True)
        if pf_idx:
            issue_idx(j + nbb, bb + nbb)
        if pf_gather:
            issue_gather(j + nb, bb + nb)

    ngrp = n // dd
    for j in range(nbb):
        issue_idx(j, j)
    for j in range(nb):
        issue_gather(j, j)
    for bb in range(dd):
        process(bb, bb, bb >= nb, bb + nbb < n, bb + nb < n)

    def grp(g, _):
        for bb in range(dd):
            process(g * dd + bb, bb, True, True, True)
        return _

    lax.fori_loop(1, ngrp - 1, grp, 0)
    for bb in range(dd):
        j = (ngrp - 1) * dd + bb
        process(j, bb, True, j + nbb < n, j + nb < n)
    for t in range(n - ngrp * dd):
        j = ngrp * dd + t
        process(j, t, True, j + nbb < n, j + nb < n)
    for j in range(n - nb, n):
        wait_scatter(j % nbb)


def _sweep_scratch(nb, ce, width):
    return [
        pltpu.VMEM((2 * nb, ce), jnp.int32),
        pltpu.VMEM((4 * nb, ce), jnp.int32),
        pltpu.VMEM((2 * nb, ce, width), jnp.float32),
    ] + [pltpu.SemaphoreType.DMA] * (6 * nb)


NB1, CE1 = 3, 40
NB2, CE2 = 3, 80


def _sc_pass1(xaug, src, dst, zeros1):
    n_chunks = E // (NC * NS) // CE1

    @functools.partial(
        pl.kernel,
        out_type=jax.ShapeDtypeStruct((NC, N, C1), jnp.float32),
        mesh=plsc.VectorSubcoreMesh(core_axis_name="c", subcore_axis_name="s"),
        compiler_params=pltpu.CompilerParams(use_tc_tiling_on_sc=False),
        scratch_types=_sweep_scratch(NB1, CE1, C1) + [pltpu.VMEM_SHARED((N, C1), jnp.float32)],
    )
    def k(xaug_h, src_h, dst_h, zeros_h, out_h, srcb, dstb, rows, *rest):
        sems, acc = list(rest[:6 * NB1]), rest[6 * NB1]
        c = lax.axis_index("c")
        s = lax.axis_index("s")
        _zero_and_barrier(zeros_h, acc, s)
        tid = c * NS + s
        edge_base = tid * (E // (NC * NS))
        _emit_sweep(
            n_chunks, NB1, CE1,
            lambda j: src_h.at[pl.ds(edge_base + j * CE1, CE1)],
            lambda j: dst_h.at[pl.ds(edge_base + j * CE1, CE1)],
            xaug_h, acc, srcb, dstb, rows, sems)
        plsc.subcore_barrier()
        _writeback(acc, out_h, c, s)

    return k(xaug, src, dst, zeros1)


def _sc_pass2(tbl_flat, src4, dst, zeros2):
    n_chunks = E // NS // CE2

    @functools.partial(
        pl.kernel,
        out_type=jax.ShapeDtypeStruct((2 * NC, N, C2), jnp.float32),
        mesh=plsc.VectorSubcoreMesh(core_axis_name="c", subcore_axis_name="s"),
        compiler_params=pltpu.CompilerParams(use_tc_tiling_on_sc=False),
        scratch_types=_sweep_scratch(NB2, CE2, C2) + [pltpu.VMEM_SHARED((N, C2), jnp.float32)],
    )
    def k(tbl_h, src4_h, dst_h, zeros_h, out_h, srcb, dstb, rows, *rest):
        sems, acc = list(rest[:6 * NB2]), rest[6 * NB2]
        c = lax.axis_index("c")
        s = lax.axis_index("s")
        edge_base = s * (E // NS)
        for p in range(2):
            q = p * NC + c
            _zero_and_barrier(zeros_h, acc, s)
            _emit_sweep(
                n_chunks, NB2, CE2,
                lambda j: src4_h.at[q, pl.ds(edge_base + j * CE2, CE2)],
                lambda j: dst_h.at[pl.ds(edge_base + j * CE2, CE2)],
                tbl_h, acc, srcb, dstb, rows, sems)
            plsc.subcore_barrier()
            _writeback(acc, out_h, q, s)

    return k(tbl_flat, src4, dst, zeros2)


RB = 2000


def _stage_b_body(s1p, xb, wla, bla, wra, wls, bls, wrs, wlg, blg, wrg,
                  ha, hs, hg, emb2, rcnt):
    s1 = s1p[0] + s1p[1]
    cnt = s1[:, D:D + 1]
    rc = 1.0 / jnp.maximum(cnt, 1.0)
    agg = s1[:, :D] * rc
    x = xb[...]

    def head(wl, bl, wr):
        return (jnp.dot(agg, wl[...], preferred_element_type=jnp.float32)
                + bl[...]
                + jnp.dot(x, wr[...], preferred_element_type=jnp.float32))

    h_a = head(wla, bla, wra)
    h_s = head(wls, bls, wrs)
    h_g = head(wlg, blg, wrg)
    ha[...] = h_a
    hs[...] = h_s
    hg[...] = h_g
    full_e = jnp.maximum(jnp.concatenate([h_a, h_s, h_g], axis=1), 0.0)
    for q in range(4):
        emb2[q] = full_e[:, q * C2:(q + 1) * C2]
    rcnt[...] = jnp.broadcast_to(rc, (RB, 8))


def _tc_stage_b(s1p, x, wla, bla, wra, wls, bls, wrs, wlg, blg, wrg):
    grid = (N // RB,)
    full = lambda shape: pl.BlockSpec(shape, lambda i: (0,) * len(shape))
    row = lambda w: pl.BlockSpec((RB, w), lambda i: (i, 0))
    return pl.pallas_call(
        _stage_b_body,
        grid=grid,
        in_specs=[
            pl.BlockSpec((NC, RB, C1), lambda i: (0, i, 0)),
            row(D),
            full((D, H)), full((1, H)), full((D, H)),
            full((D, H)), full((1, H)), full((D, H)),
            full((D, H)), full((1, H)), full((D, H)),
        ],
        out_specs=[
            row(H), row(H), row(H),
            pl.BlockSpec((2 * NC, RB, C2), lambda i: (0, i, 0)),
            row(8),
        ],
        out_shape=[
            jax.ShapeDtypeStruct((N, H), jnp.float32),
            jax.ShapeDtypeStruct((N, H), jnp.float32),
            jax.ShapeDtypeStruct((N, H), jnp.float32),
            jax.ShapeDtypeStruct((2 * NC, N, C2), jnp.float32),
            jax.ShapeDtypeStruct((N, 8), jnp.float32),
        ],
    )(s1p, x, wla, bla, wra, wls, bls, wrs, wlg, blg, wrg)


def _log_softmax(xo):
    m = jnp.max(xo, axis=1, keepdims=True)
    e = jnp.exp(xo - m)
    return xo - m - jnp.log(jnp.sum(e, axis=1, keepdims=True))


def _stage_d_body(s2, rcnt, emb2, wa, ba, ra, ws, bs, rs, wg, bg, rg,
                  ya, ys, yg):
    rc = rcnt[:, :1]
    s2_full = jnp.concatenate([s2[q] for q in range(4)], axis=1)
    e_full = jnp.concatenate([emb2[q] for q in range(4)], axis=1)
    agg_a = s2_full[:, :H] * rc
    agg_s = s2_full[:, H:2 * H] * rc
    agg_g = s2_full[:, 2 * H:] * rc
    e_a = e_full[:, :H]
    e_s = e_full[:, H:2 * H]
    e_g = e_full[:, 2 * H:]

    def head(agg, emb, wl, bl, wr):
        return (jnp.dot(agg, wl[...], preferred_element_type=jnp.float32)
                + bl[...]
                + jnp.dot(emb, wr[...], preferred_element_type=jnp.float32))

    ya[...] = _log_softmax(head(agg_a, e_a, wa, ba, ra))
    ys[...] = _log_softmax(head(agg_s, e_s, ws, bs, rs))
    yg[...] = _log_softmax(head(agg_g, e_g, wg, bg, rg))


def _tc_stage_d(s2, rcnt, emb2, wa, ba, ra, ws, bs, rs, wg, bg, rg):
    grid = (N // RB,)
    full = lambda shape: pl.BlockSpec(shape, lambda i: (0,) * len(shape))
    row = lambda w: pl.BlockSpec((RB, w), lambda i: (i, 0))
    plane = pl.BlockSpec((2 * NC, RB, C2), lambda i: (0, i, 0))
    oa, os_, og = wa.shape[1], ws.shape[1], wg.shape[1]
    return pl.pallas_call(
        _stage_d_body,
        grid=grid,
        in_specs=[
            plane, row(8), plane,
            full((H, oa)), full((1, oa)), full((H, oa)),
            full((H, os_)), full((1, os_)), full((H, os_)),
            full((H, og)), full((1, og)), full((H, og)),
        ],
        out_specs=[row(oa), row(os_), row(og)],
        out_shape=[
            jax.ShapeDtypeStruct((N, oa), jnp.float32),
            jax.ShapeDtypeStruct((N, os_), jnp.float32),
            jax.ShapeDtypeStruct((N, og), jnp.float32),
        ],
    )(s2, rcnt, emb2, wa, ba, ra, ws, bs, rs, wg, bg, rg)


def _pad_out_head(w, b, wr, padded):
    oc = w.shape[1]
    wp = jnp.pad(w, ((0, 0), (0, padded - oc)))
    rp = jnp.pad(wr, ((0, 0), (0, padded - oc)))
    bp = jnp.pad(b.reshape(1, -1), ((0, 0), (0, padded - oc)),
                 constant_values=-1e30)
    return wp, bp, rp


def kernel(x, edge_index,
           Wl1_artist, bl1_artist, Wr1_artist, Wlo_artist, blo_artist, Wro_artist,
           Wl1_style, bl1_style, Wr1_style, Wlo_style, blo_style, Wro_style,
           Wl1_genre, bl1_genre, Wr1_genre, Wlo_genre, blo_genre, Wro_genre):
    src = edge_index[0].astype(jnp.int32)
    dst = edge_index[1].astype(jnp.int32)

    xaug = jnp.concatenate(
        [x, jnp.ones((N, 1), jnp.float32), jnp.zeros((N, C1 - D - 1), jnp.float32)],
        axis=1)
    zeros1 = jnp.zeros((N, C1), jnp.float32)
    zeros2 = jnp.zeros((N, C2), jnp.float32)
    src4 = src[None, :] + (jnp.arange(4, dtype=jnp.int32) * N)[:, None]

    s1p = _sc_pass1(xaug, src, dst, zeros1)

    ha, hs, hg, emb2, rcnt = _tc_stage_b(
        s1p, x,
        Wl1_artist, bl1_artist.reshape(1, -1), Wr1_artist,
        Wl1_style, bl1_style.reshape(1, -1), Wr1_style,
        Wl1_genre, bl1_genre.reshape(1, -1), Wr1_genre)

    s2 = _sc_pass2(emb2.reshape(4 * N, C2), src4, dst, zeros2)

    wa, ba, ra = _pad_out_head(Wlo_artist, blo_artist, Wro_artist, 256)
    ws, bs, rs = _pad_out_head(Wlo_style, blo_style, Wro_style, 128)
    wg, bg, rg = _pad_out_head(Wlo_genre, blo_genre, Wro_genre, 128)

    ya, ys, yg = _tc_stage_d(s2, rcnt, emb2, wa, ba, ra, ws, bs, rs, wg, bg, rg)

    return (ha, ya[:, :Wlo_artist.shape[1]],
            hs, ys[:, :Wlo_style.shape[1]],
            hg, yg[:, :Wlo_genre.shape[1]])


# speedup vs baseline: 11.5881x; 1.3918x over previous
import functools

import jax
import jax.numpy as jnp
from jax import lax
from jax.experimental import pallas as pl
from jax.experimental.pallas import tpu as pltpu
from jax.experimental.pallas import tpu_sc as plsc

N = 10000
E = 320000
D = 128
H = 128

NC = 2
NS = 16
CE = 80

C1 = 144
C2 = 96

ZR = 624
ZTAIL_BASE = ZR * 15
ZTAIL = N - ZTAIL_BASE - ZR


def _zero_and_barrier(zeros_hbm, acc, s):
    pltpu.sync_copy(zeros_hbm.at[pl.ds(s * ZR, ZR)], acc.at[pl.ds(s * ZR, ZR)])

    @pl.when(s == NS - 1)
    def _():
        pltpu.sync_copy(zeros_hbm.at[pl.ds(ZTAIL_BASE + ZR, ZTAIL)],
                        acc.at[pl.ds(ZTAIL_BASE + ZR, ZTAIL)])

    plsc.subcore_barrier()


def _writeback(acc, out_hbm, c, s):
    pltpu.sync_copy(acc.at[pl.ds(s * ZR, ZR)], out_hbm.at[c, pl.ds(s * ZR, ZR)])

    @pl.when(s == NS - 1)
    def _():
        pltpu.sync_copy(acc.at[pl.ds(ZTAIL_BASE + ZR, ZTAIL)],
                        out_hbm.at[c, pl.ds(ZTAIL_BASE + ZR, ZTAIL)])


def _emit_sweep(n, nb, ce, src_ix, dst_ix, tbl, acc, srcb, dstb, rows, sems):
    nbb = 2 * nb
    dd = 2 * nbb
    gsems = sems[:nb]
    ssems = sems[nb:2 * nb]
    isems = sems[2 * nb:2 * nb + nbb]
    dsems = sems[2 * nb + nbb:]

    def issue_idx(j, bb):
        sb, db = bb % nbb, bb % dd
        pltpu.async_copy(src_ix(j), srcb.at[sb], isems[sb])
        pltpu.async_copy(dst_ix(j), dstb.at[db], dsems[sb])

    def issue_gather(j, bb):
        b, sb, rb = bb % nb, bb % nbb, bb % nbb
        pltpu.make_async_copy(src_ix(j), srcb.at[sb], isems[sb]).wait()
        pltpu.async_copy(tbl.at[srcb.at[sb]], rows.at[rb], gsems[b])

    def wait_scatter(bb):
        b, rb = bb % nb, bb % nbb
        pltpu.make_async_copy(rows.at[rb], acc.at[dstb.at[0]], ssems[b]).wait()

    def process(j, bb, wait_prev, pf_idx, pf_gather):
        b, sb, rb, db = bb % nb, bb % nbb, bb % nbb, bb % dd
        if wait_prev:
            wait_scatter(bb + nb)
        pltpu.make_async_copy(tbl.at[srcb.at[sb]], rows.at[rb], gsems[b]).wait()
        pltpu.make_async_copy(dst_ix(j), dstb.at[db], dsems[sb]).wait()
        pltpu.async_copy(rows.at[rb], acc.at[dstb.at[db]], ssems[b], add=True)
        if pf_idx:
            issue_idx(j + nbb, bb + nbb)
        if pf_gather:
            issue_gather(j + nb, bb + nb)

    ngrp = n // dd
    for j in range(nbb):
        issue_idx(j, j)
    for j in range(nb):
        issue_gather(j, j)
    for bb in range(dd):
        process(bb, bb, bb >= nb, bb + nbb < n, bb + nb < n)

    def grp(g, _):
        for bb in range(dd):
            process(g * dd + bb, bb, True, True, True)
        return _

    lax.fori_loop(1, ngrp - 1, grp, 0)
    for bb in range(dd):
        j = (ngrp - 1) * dd + bb
        process(j, bb, True, j + nbb < n, j + nb < n)
    for t in range(n - ngrp * dd):
        j = ngrp * dd + t
        process(j, t, True, j + nbb < n, j + nb < n)
    for j in range(n - nb, n):
        wait_scatter(j % nbb)


def _sweep_scratch(nb, ce, width):
    return [
        pltpu.VMEM((2 * nb, ce), jnp.int32),
        pltpu.VMEM((4 * nb, ce), jnp.int32),
        pltpu.VMEM((2 * nb, ce, width), jnp.float32),
    ] + [pltpu.SemaphoreType.DMA] * (6 * nb)


NB1, CE1 = 3, 40
NB2, CE2 = 3, 80


def _sc_pass1(xaug, src, dst, zeros1):
    n_chunks = E // (NC * NS) // CE1

    @functools.partial(
        pl.kernel,
        out_type=jax.ShapeDtypeStruct((NC, N, C1), jnp.float32),
        mesh=plsc.VectorSubcoreMesh(core_axis_name="c", subcore_axis_name="s"),
        compiler_params=pltpu.CompilerParams(use_tc_tiling_on_sc=False),
        scratch_types=_sweep_scratch(NB1, CE1, C1) + [pltpu.VMEM_SHARED((N, C1), jnp.float32)],
    )
    def k(xaug_h, src_h, dst_h, zeros_h, out_h, srcb, dstb, rows, *rest):
        sems, acc = list(rest[:6 * NB1]), rest[6 * NB1]
        c = lax.axis_index("c")
        s = lax.axis_index("s")
        _zero_and_barrier(zeros_h, acc, s)
        tid = c * NS + s
        edge_base = tid * (E // (NC * NS))
        _emit_sweep(
            n_chunks, NB1, CE1,
            lambda j: src_h.at[pl.ds(edge_base + j * CE1, CE1)],
            lambda j: dst_h.at[pl.ds(edge_base + j * CE1, CE1)],
            xaug_h, acc, srcb, dstb, rows, sems)
        plsc.subcore_barrier()
        _writeback(acc, out_h, c, s)

    return k(xaug, src, dst, zeros1)


def _sc_pass2(z_flat, src2, dst, zeros2):
    n_chunks = E // NS // CE2

    @functools.partial(
        pl.kernel,
        out_type=jax.ShapeDtypeStruct((NC, N, C2), jnp.float32),
        mesh=plsc.VectorSubcoreMesh(core_axis_name="c", subcore_axis_name="s"),
        compiler_params=pltpu.CompilerParams(use_tc_tiling_on_sc=False),
        scratch_types=_sweep_scratch(NB2, CE2, C2) + [pltpu.VMEM_SHARED((N, C2), jnp.float32)],
    )
    def k(z_h, src2_h, dst_h, zeros_h, out_h, srcb, dstb, rows, *rest):
        sems, acc = list(rest[:6 * NB2]), rest[6 * NB2]
        c = lax.axis_index("c")
        s = lax.axis_index("s")
        edge_base = s * (E // NS)
        _zero_and_barrier(zeros_h, acc, s)
        _emit_sweep(
            n_chunks, NB2, CE2,
            lambda j: src2_h.at[c, pl.ds(edge_base + j * CE2, CE2)],
            lambda j: dst_h.at[pl.ds(edge_base + j * CE2, CE2)],
            z_h, acc, srcb, dstb, rows, sems)
        plsc.subcore_barrier()
        _writeback(acc, out_h, c, s)

    return k(z_flat, src2, dst, zeros2)


RB = 2000

OA, OS, OG = 129, 27, 11
OZ = OA + OS + OG


def _stage_b_body(s1p, xb, wla, bla, wra, wls, bls, wrs, wlg, blg, wrg,
                  woa, wos, wog, ha, hs, hg, z2, rcnt):
    s1 = s1p[0] + s1p[1]
    cnt = s1[:, D:D + 1]
    rc = 1.0 / jnp.maximum(cnt, 1.0)
    agg = s1[:, :D] * rc
    x = xb[...]

    def head(wl, bl, wr):
        return (jnp.dot(agg, wl[...], preferred_element_type=jnp.float32)
                + bl[...]
                + jnp.dot(x, wr[...], preferred_element_type=jnp.float32))

    h_a = head(wla, bla, wra)
    h_s = head(wls, bls, wrs)
    h_g = head(wlg, blg, wrg)
    ha[...] = h_a
    hs[...] = h_s
    hg[...] = h_g
    z = jnp.concatenate(
        [jnp.dot(jnp.maximum(h_a, 0.0), woa[...], preferred_element_type=jnp.float32),
         jnp.dot(jnp.maximum(h_s, 0.0), wos[...], preferred_element_type=jnp.float32),
         jnp.dot(jnp.maximum(h_g, 0.0), wog[...], preferred_element_type=jnp.float32),
         jnp.zeros((RB, 2 * C2 - OZ), jnp.float32)],
        axis=1)
    z2[0] = z[:, :C2]
    z2[1] = z[:, C2:]
    rcnt[...] = jnp.broadcast_to(rc, (RB, 8))


def _tc_stage_b(s1p, x, wla, bla, wra, wls, bls, wrs, wlg, blg, wrg,
                woa, wos, wog):
    grid = (N // RB,)
    full = lambda shape: pl.BlockSpec(shape, lambda i: (0,) * len(shape))
    row = lambda w: pl.BlockSpec((RB, w), lambda i: (i, 0))
    return pl.pallas_call(
        _stage_b_body,
        grid=grid,
        in_specs=[
            pl.BlockSpec((NC, RB, C1), lambda i: (0, i, 0)),
            row(D),
            full((D, H)), full((1, H)), full((D, H)),
            full((D, H)), full((1, H)), full((D, H)),
            full((D, H)), full((1, H)), full((D, H)),
            full((H, OA)), full((H, OS)), full((H, OG)),
        ],
        out_specs=[
            row(H), row(H), row(H),
            pl.BlockSpec((NC, RB, C2), lambda i: (0, i, 0)),
            row(8),
        ],
        out_shape=[
            jax.ShapeDtypeStruct((N, H), jnp.float32),
            jax.ShapeDtypeStruct((N, H), jnp.float32),
            jax.ShapeDtypeStruct((N, H), jnp.float32),
            jax.ShapeDtypeStruct((NC, N, C2), jnp.float32),
            jax.ShapeDtypeStruct((N, 8), jnp.float32),
        ],
    )(s1p, x, wla, bla, wra, wls, bls, wrs, wlg, blg, wrg, woa, wos, wog)


def _log_softmax(xo):
    m = jnp.max(xo, axis=1, keepdims=True)
    e = jnp.exp(xo - m)
    return xo - m - jnp.log(jnp.sum(e, axis=1, keepdims=True))


def _stage_d_body(s2, rcnt, hab, hsb, hgb, ba, ra, bs, rs, bg, rg,
                  ya, ys, yg):
    rc = rcnt[:, :1]
    zagg = jnp.concatenate([s2[0], s2[1]], axis=1) * rc

    def head(lo, hi, hb, bl, wr):
        e = jnp.maximum(hb[...], 0.0)
        return _log_softmax(
            zagg[:, lo:hi] + bl[...]
            + jnp.dot(e, wr[...], preferred_element_type=jnp.float32))

    ya[...] = head(0, OA, hab, ba, ra)
    ys[...] = head(OA, OA + OS, hsb, bs, rs)
    yg[...] = head(OA + OS, OZ, hgb, bg, rg)


def _tc_stage_d(s2, rcnt, ha, hs, hg, ba, ra, bs, rs, bg, rg):
    grid = (N // RB,)
    full = lambda shape: pl.BlockSpec(shape, lambda i: (0,) * len(shape))
    row = lambda w: pl.BlockSpec((RB, w), lambda i: (i, 0))
    return pl.pallas_call(
        _stage_d_body,
        grid=grid,
        in_specs=[
            pl.BlockSpec((NC, RB, C2), lambda i: (0, i, 0)),
            row(8), row(H), row(H), row(H),
            full((1, OA)), full((H, OA)),
            full((1, OS)), full((H, OS)),
            full((1, OG)), full((H, OG)),
        ],
        out_specs=[row(OA), row(OS), row(OG)],
        out_shape=[
            jax.ShapeDtypeStruct((N, OA), jnp.float32),
            jax.ShapeDtypeStruct((N, OS), jnp.float32),
            jax.ShapeDtypeStruct((N, OG), jnp.float32),
        ],
    )(s2, rcnt, ha, hs, hg, ba, ra, bs, rs, bg, rg)


def kernel(x, edge_index,
           Wl1_artist, bl1_artist, Wr1_artist, Wlo_artist, blo_artist, Wro_artist,
           Wl1_style, bl1_style, Wr1_style, Wlo_style, blo_style, Wro_style,
           Wl1_genre, bl1_genre, Wr1_genre, Wlo_genre, blo_genre, Wro_genre):
    src = edge_index[0].astype(jnp.int32)
    dst = edge_index[1].astype(jnp.int32)

    xaug = jnp.concatenate(
        [x, jnp.ones((N, 1), jnp.float32), jnp.zeros((N, C1 - D - 1), jnp.float32)],
        axis=1)
    zeros1 = jnp.zeros((N, C1), jnp.float32)
    zeros2 = jnp.zeros((N, C2), jnp.float32)
    src2 = src[None, :] + (jnp.arange(2, dtype=jnp.int32) * N)[:, None]

    s1p = _sc_pass1(xaug, src, dst, zeros1)

    ha, hs, hg, z2, rcnt = _tc_stage_b(
        s1p, x,
        Wl1_artist, bl1_artist.reshape(1, -1), Wr1_artist,
        Wl1_style, bl1_style.reshape(1, -1), Wr1_style,
        Wl1_genre, bl1_genre.reshape(1, -1), Wr1_genre,
        Wlo_artist, Wlo_style, Wlo_genre)

    s2 = _sc_pass2(z2.reshape(2 * N, C2), src2, dst, zeros2)

    ya, ys, yg = _tc_stage_d(
        s2, rcnt, ha, hs, hg,
        blo_artist.reshape(1, -1), Wro_artist,
        blo_style.reshape(1, -1), Wro_style,
        blo_genre.reshape(1, -1), Wro_genre)

    return (ha, ya, hs, ys, hg, yg)


# speedup vs baseline: 11.6744x; 1.0074x over previous
import functools

import jax
import jax.numpy as jnp
from jax import lax
from jax.experimental import pallas as pl
from jax.experimental.pallas import tpu as pltpu
from jax.experimental.pallas import tpu_sc as plsc

N = 10000
E = 320000
D = 128
H = 128

NC = 2
NS = 16
CE = 80

CW = 16
C2 = 96

ZR = 624
ZTAIL_BASE = ZR * 15
ZTAIL = N - ZTAIL_BASE - ZR


def _zero_and_barrier(zeros_hbm, acc, s):
    pltpu.sync_copy(zeros_hbm.at[pl.ds(s * ZR, ZR)], acc.at[pl.ds(s * ZR, ZR)])

    @pl.when(s == NS - 1)
    def _():
        pltpu.sync_copy(zeros_hbm.at[pl.ds(ZTAIL_BASE + ZR, ZTAIL)],
                        acc.at[pl.ds(ZTAIL_BASE + ZR, ZTAIL)])

    plsc.subcore_barrier()


def _writeback(acc, out_hbm, c, s):
    pltpu.sync_copy(acc.at[pl.ds(s * ZR, ZR)], out_hbm.at[c, pl.ds(s * ZR, ZR)])

    @pl.when(s == NS - 1)
    def _():
        pltpu.sync_copy(acc.at[pl.ds(ZTAIL_BASE + ZR, ZTAIL)],
                        out_hbm.at[c, pl.ds(ZTAIL_BASE + ZR, ZTAIL)])


def _emit_sweep(n, nb, ce, src_ix, dst_ix, tbl, acc, srcb, dstb, rows, sems,
                cnt=None):
    nbb = 2 * nb
    dd = 2 * nbb
    gsems = sems[:nb]
    ssems = sems[nb:2 * nb]
    isems = sems[2 * nb:2 * nb + nbb]
    dsems = sems[2 * nb + nbb:6 * nb]
    if cnt is not None:
        acc_cnt, ones_ref = cnt
        csems = sems[6 * nb:]

    def issue_idx(j, bb):
        sb, db = bb % nbb, bb % dd
        pltpu.async_copy(src_ix(j), srcb.at[sb], isems[sb])
        pltpu.async_copy(dst_ix(j), dstb.at[db], dsems[sb])

    def issue_gather(j, bb):
        b, sb, rb = bb % nb, bb % nbb, bb % nbb
        pltpu.make_async_copy(src_ix(j), srcb.at[sb], isems[sb]).wait()
        pltpu.async_copy(tbl.at[srcb.at[sb]], rows.at[rb], gsems[b])

    def wait_scatter(bb):
        b, rb = bb % nb, bb % nbb
        pltpu.make_async_copy(rows.at[rb], acc.at[dstb.at[0]], ssems[b]).wait()
        if cnt is not None:
            pltpu.make_async_copy(ones_ref, acc_cnt.at[dstb.at[0]], csems[b]).wait()

    def process(j, bb, wait_prev, pf_idx, pf_gather):
        b, sb, rb, db = bb % nb, bb % nbb, bb % nbb, bb % dd
        if wait_prev:
            wait_scatter(bb + nb)
        pltpu.make_async_copy(tbl.at[srcb.at[sb]], rows.at[rb], gsems[b]).wait()
        pltpu.make_async_copy(dst_ix(j), dstb.at[db], dsems[sb]).wait()
        pltpu.async_copy(rows.at[rb], acc.at[dstb.at[db]], ssems[b], add=True)
        if cnt is not None:
            pltpu.async_copy(ones_ref, acc_cnt.at[dstb.at[db]], csems[b], add=True)
        if pf_idx:
            issue_idx(j + nbb, bb + nbb)
        if pf_gather:
            issue_gather(j + nb, bb + nb)

    ngrp = n // dd
    for j in range(nbb):
        issue_idx(j, j)
    for j in range(nb):
        issue_gather(j, j)
    for bb in range(dd):
        process(bb, bb, bb >= nb, bb + nbb < n, bb + nb < n)

    def grp(g, _):
        for bb in range(dd):
            process(g * dd + bb, bb, True, True, True)
        return _

    lax.fori_loop(1, ngrp - 1, grp, 0)
    for bb in range(dd):
        j = (ngrp - 1) * dd + bb
        process(j, bb, True, j + nbb < n, j + nb < n)
    for t in range(n - ngrp * dd):
        j = ngrp * dd + t
        process(j, t, True, j + nbb < n, j + nb < n)
    for j in range(n - nb, n):
        wait_scatter(j % nbb)


def _sweep_scratch(nb, ce, width):
    return [
        pltpu.VMEM((2 * nb, ce), jnp.int32),
        pltpu.VMEM((4 * nb, ce), jnp.int32),
        pltpu.VMEM((2 * nb, ce, width), jnp.float32),
    ] + [pltpu.SemaphoreType.DMA] * (6 * nb)


NB1, CE1 = 2, 40
NB2, CE2 = 3, 80


def _sc_pass1(x, src, dst, zeros1, zerosc, ones16):
    n_chunks = E // (NC * NS) // CE1

    @functools.partial(
        pl.kernel,
        out_type=[jax.ShapeDtypeStruct((NC, N, D), jnp.float32),
                  jax.ShapeDtypeStruct((NC, N, CW), jnp.float32)],
        mesh=plsc.VectorSubcoreMesh(core_axis_name="c", subcore_axis_name="s"),
        compiler_params=pltpu.CompilerParams(use_tc_tiling_on_sc=False),
        scratch_types=_sweep_scratch(NB1, CE1, D)
        + [pltpu.SemaphoreType.DMA] * NB1
        + [pltpu.VMEM((CE1, CW), jnp.float32),
           pltpu.VMEM_SHARED((N, D), jnp.float32),
           pltpu.VMEM_SHARED((N, CW), jnp.float32)],
    )
    def k(x_h, src_h, dst_h, zeros_h, zc_h, ones_h, out_h, outc_h,
          srcb, dstb, rows, *rest):
        sems = list(rest[:7 * NB1])
        onesb, acc, acc_cnt = rest[7 * NB1], rest[7 * NB1 + 1], rest[7 * NB1 + 2]
        c = lax.axis_index("c")
        s = lax.axis_index("s")
        pltpu.sync_copy(ones_h, onesb)
        pltpu.sync_copy(zc_h.at[pl.ds(s * ZR, ZR)], acc_cnt.at[pl.ds(s * ZR, ZR)])

        @pl.when(s == NS - 1)
        def _():
            pltpu.sync_copy(zc_h.at[pl.ds(ZTAIL_BASE + ZR, ZTAIL)],
                            acc_cnt.at[pl.ds(ZTAIL_BASE + ZR, ZTAIL)])

        _zero_and_barrier(zeros_h, acc, s)
        tid = c * NS + s
        edge_base = tid * (E // (NC * NS))
        _emit_sweep(
            n_chunks, NB1, CE1,
            lambda j: src_h.at[pl.ds(edge_base + j * CE1, CE1)],
            lambda j: dst_h.at[pl.ds(edge_base + j * CE1, CE1)],
            x_h, acc, srcb, dstb, rows, sems, cnt=(acc_cnt, onesb))
        plsc.subcore_barrier()
        _writeback(acc, out_h, c, s)
        pltpu.sync_copy(acc_cnt.at[pl.ds(s * ZR, ZR)], outc_h.at[c, pl.ds(s * ZR, ZR)])

        @pl.when(s == NS - 1)
        def _():
            pltpu.sync_copy(acc_cnt.at[pl.ds(ZTAIL_BASE + ZR, ZTAIL)],
                            outc_h.at[c, pl.ds(ZTAIL_BASE + ZR, ZTAIL)])

    return k(x, src, dst, zeros1, zerosc, ones16)


def _sc_pass2(z_flat, src2, dst, zeros2):
    n_chunks = E // NS // CE2

    @functools.partial(
        pl.kernel,
        out_type=jax.ShapeDtypeStruct((NC, N, C2), jnp.float32),
        mesh=plsc.VectorSubcoreMesh(core_axis_name="c", subcore_axis_name="s"),
        compiler_params=pltpu.CompilerParams(use_tc_tiling_on_sc=False),
        scratch_types=_sweep_scratch(NB2, CE2, C2) + [pltpu.VMEM_SHARED((N, C2), jnp.float32)],
    )
    def k(z_h, src2_h, dst_h, zeros_h, out_h, srcb, dstb, rows, *rest):
        sems, acc = list(rest[:6 * NB2]), rest[6 * NB2]
        c = lax.axis_index("c")
        s = lax.axis_index("s")
        edge_base = s * (E // NS)
        _zero_and_barrier(zeros_h, acc, s)
        _emit_sweep(
            n_chunks, NB2, CE2,
            lambda j: src2_h.at[c, pl.ds(edge_base + j * CE2, CE2)],
            lambda j: dst_h.at[pl.ds(edge_base + j * CE2, CE2)],
            z_h, acc, srcb, dstb, rows, sems)
        plsc.subcore_barrier()
        _writeback(acc, out_h, c, s)

    return k(z_flat, src2, dst, zeros2)


RB = 2000

OA, OS, OG = 129, 27, 11
OZ = OA + OS + OG


def _stage_b_body(s1p, cntp, xb, wla, bla, wra, wls, bls, wrs, wlg, blg, wrg,
                  woa, wos, wog, ha, hs, hg, z2, rcnt):
    cnt = cntp[0][:, :1] + cntp[1][:, :1]
    rc = 1.0 / jnp.maximum(cnt, 1.0)
    agg = (s1p[0] + s1p[1]) * rc
    x = xb[...]

    def head(wl, bl, wr):
        return (jnp.dot(agg, wl[...], preferred_element_type=jnp.float32)
                + bl[...]
                + jnp.dot(x, wr[...], preferred_element_type=jnp.float32))

    h_a = head(wla, bla, wra)
    h_s = head(wls, bls, wrs)
    h_g = head(wlg, blg, wrg)
    ha[...] = h_a
    hs[...] = h_s
    hg[...] = h_g
    z = jnp.concatenate(
        [jnp.dot(jnp.maximum(h_a, 0.0), woa[...], preferred_element_type=jnp.float32),
         jnp.dot(jnp.maximum(h_s, 0.0), wos[...], preferred_element_type=jnp.float32),
         jnp.dot(jnp.maximum(h_g, 0.0), wog[...], preferred_element_type=jnp.float32),
         jnp.zeros((RB, 2 * C2 - OZ), jnp.float32)],
        axis=1)
    z2[0] = z[:, :C2]
    z2[1] = z[:, C2:]
    rcnt[...] = jnp.broadcast_to(rc, (RB, 8))


def _tc_stage_b(s1p, cntp, x, wla, bla, wra, wls, bls, wrs, wlg, blg, wrg,
                woa, wos, wog):
    grid = (N // RB,)
    full = lambda shape: pl.BlockSpec(shape, lambda i: (0,) * len(shape))
    row = lambda w: pl.BlockSpec((RB, w), lambda i: (i, 0))
    return pl.pallas_call(
        _stage_b_body,
        grid=grid,
        in_specs=[
            pl.BlockSpec((NC, RB, D), lambda i: (0, i, 0)),
            pl.BlockSpec((NC, RB, CW), lambda i: (0, i, 0)),
            row(D),
            full((D, H)), full((1, H)), full((D, H)),
            full((D, H)), full((1, H)), full((D, H)),
            full((D, H)), full((1, H)), full((D, H)),
            full((H, OA)), full((H, OS)), full((H, OG)),
        ],
        out_specs=[
            row(H), row(H), row(H),
            pl.BlockSpec((NC, RB, C2), lambda i: (0, i, 0)),
            row(8),
        ],
        out_shape=[
            jax.ShapeDtypeStruct((N, H), jnp.float32),
            jax.ShapeDtypeStruct((N, H), jnp.float32),
            jax.ShapeDtypeStruct((N, H), jnp.float32),
            jax.ShapeDtypeStruct((NC, N, C2), jnp.float32),
            jax.ShapeDtypeStruct((N, 8), jnp.float32),
        ],
    )(s1p, cntp, x, wla, bla, wra, wls, bls, wrs, wlg, blg, wrg, woa, wos, wog)


def _log_softmax(xo):
    m = jnp.max(xo, axis=1, keepdims=True)
    e = jnp.exp(xo - m)
    return xo - m - jnp.log(jnp.sum(e, axis=1, keepdims=True))


def _stage_d_body(s2, rcnt, hab, hsb, hgb, ba, ra, bs, rs, bg, rg,
                  ya, ys, yg):
    rc = rcnt[:, :1]
    zagg = jnp.concatenate([s2[0], s2[1]], axis=1) * rc

    def head(lo, hi, hb, bl, wr):
        e = jnp.maximum(hb[...], 0.0)
        return _log_softmax(
            zagg[:, lo:hi] + bl[...]
            + jnp.dot(e, wr[...], preferred_element_type=jnp.float32))

    ya[...] = head(0, OA, hab, ba, ra)
    ys[...] = head(OA, OA + OS, hsb, bs, rs)
    yg[...] = head(OA + OS, OZ, hgb, bg, rg)


def _tc_stage_d(s2, rcnt, ha, hs, hg, ba, ra, bs, rs, bg, rg):
    grid = (N // RB,)
    full = lambda shape: pl.BlockSpec(shape, lambda i: (0,) * len(shape))
    row = lambda w: pl.BlockSpec((RB, w), lambda i: (i, 0))
    return pl.pallas_call(
        _stage_d_body,
        grid=grid,
        in_specs=[
            pl.BlockSpec((NC, RB, C2), lambda i: (0, i, 0)),
            row(8), row(H), row(H), row(H),
            full((1, OA)), full((H, OA)),
            full((1, OS)), full((H, OS)),
            full((1, OG)), full((H, OG)),
        ],
        out_specs=[row(OA), row(OS), row(OG)],
        out_shape=[
            jax.ShapeDtypeStruct((N, OA), jnp.float32),
            jax.ShapeDtypeStruct((N, OS), jnp.float32),
            jax.ShapeDtypeStruct((N, OG), jnp.float32),
        ],
    )(s2, rcnt, ha, hs, hg, ba, ra, bs, rs, bg, rg)


def kernel(x, edge_index,
           Wl1_artist, bl1_artist, Wr1_artist, Wlo_artist, blo_artist, Wro_artist,
           Wl1_style, bl1_style, Wr1_style, Wlo_style, blo_style, Wro_style,
           Wl1_genre, bl1_genre, Wr1_genre, Wlo_genre, blo_genre, Wro_genre):
    src = edge_index[0].astype(jnp.int32)
    dst = edge_index[1].astype(jnp.int32)

    zeros1 = jnp.zeros((N, D), jnp.float32)
    zerosc = jnp.zeros((N, CW), jnp.float32)
    zeros2 = jnp.zeros((N, C2), jnp.float32)
    ones16 = jnp.ones((CE1, CW), jnp.float32)
    src2 = src[None, :] + (jnp.arange(2, dtype=jnp.int32) * N)[:, None]

    s1p, cntp = _sc_pass1(x, src, dst, zeros1, zerosc, ones16)

    ha, hs, hg, z2, rcnt = _tc_stage_b(
        s1p, cntp, x,
        Wl1_artist, bl1_artist.reshape(1, -1), Wr1_artist,
        Wl1_style, bl1_style.reshape(1, -1), Wr1_style,
        Wl1_genre, bl1_genre.reshape(1, -1), Wr1_genre,
        Wlo_artist, Wlo_style, Wlo_genre)

    s2 = _sc_pass2(z2.reshape(2 * N, C2), src2, dst, zeros2)

    ya, ys, yg = _tc_stage_d(
        s2, rcnt, ha, hs, hg,
        blo_artist.reshape(1, -1), Wro_artist,
        blo_style.reshape(1, -1), Wro_style,
        blo_genre.reshape(1, -1), Wro_genre)

    return (ha, ya, hs, ys, hg, yg)


# speedup vs baseline: 12.8428x; 1.1001x over previous
import functools

import jax
import jax.numpy as jnp
from jax import lax
from jax.experimental import pallas as pl
from jax.experimental.pallas import tpu as pltpu
from jax.experimental.pallas import tpu_sc as plsc

N = 10000
E = 320000
D = 128
H = 128

NC = 2
NS = 16
CE = 80

CW = 16
C2 = 88

ZR = 624
ZTAIL_BASE = ZR * 15
ZTAIL = N - ZTAIL_BASE - ZR


def _zero_and_barrier(zeros_hbm, acc, s):
    pltpu.sync_copy(zeros_hbm.at[pl.ds(s * ZR, ZR)], acc.at[pl.ds(s * ZR, ZR)])

    @pl.when(s == NS - 1)
    def _():
        pltpu.sync_copy(zeros_hbm.at[pl.ds(ZTAIL_BASE + ZR, ZTAIL)],
                        acc.at[pl.ds(ZTAIL_BASE + ZR, ZTAIL)])

    plsc.subcore_barrier()


def _writeback(acc, out_hbm, c, s):
    pltpu.sync_copy(acc.at[pl.ds(s * ZR, ZR)], out_hbm.at[c, pl.ds(s * ZR, ZR)])

    @pl.when(s == NS - 1)
    def _():
        pltpu.sync_copy(acc.at[pl.ds(ZTAIL_BASE + ZR, ZTAIL)],
                        out_hbm.at[c, pl.ds(ZTAIL_BASE + ZR, ZTAIL)])


def _emit_sweep(n, nb, ce, src_ix, dst_ix, tbl, acc, srcb, dstb, rows, sems,
                cnt=None):
    nbb = 2 * nb
    dd = 2 * nbb
    gsems = sems[:nb]
    ssems = sems[nb:2 * nb]
    isems = sems[2 * nb:2 * nb + nbb]
    dsems = sems[2 * nb + nbb:6 * nb]
    if cnt is not None:
        acc_cnt, ones_ref = cnt
        csems = sems[6 * nb:]

    def issue_idx(j, bb):
        sb, db = bb % nbb, bb % dd
        pltpu.async_copy(src_ix(j), srcb.at[sb], isems[sb])
        pltpu.async_copy(dst_ix(j), dstb.at[db], dsems[sb])

    def issue_gather(j, bb):
        b, sb, rb = bb % nb, bb % nbb, bb % nbb
        pltpu.make_async_copy(src_ix(j), srcb.at[sb], isems[sb]).wait()
        pltpu.async_copy(tbl.at[srcb.at[sb]], rows.at[rb], gsems[b])

    def wait_scatter(bb):
        b, rb = bb % nb, bb % nbb
        pltpu.make_async_copy(rows.at[rb], acc.at[dstb.at[0]], ssems[b]).wait()
        if cnt is not None:
            pltpu.make_async_copy(ones_ref, acc_cnt.at[dstb.at[0]], csems[b]).wait()

    def process(j, bb, wait_prev, pf_idx, pf_gather):
        b, sb, rb, db = bb % nb, bb % nbb, bb % nbb, bb % dd
        if wait_prev:
            wait_scatter(bb + nb)
        pltpu.make_async_copy(tbl.at[srcb.at[sb]], rows.at[rb], gsems[b]).wait()
        pltpu.make_async_copy(dst_ix(j), dstb.at[db], dsems[sb]).wait()
        pltpu.async_copy(rows.at[rb], acc.at[dstb.at[db]], ssems[b], add=True)
        if cnt is not None:
            pltpu.async_copy(ones_ref, acc_cnt.at[dstb.at[db]], csems[b], add=True)
        if pf_idx:
            issue_idx(j + nbb, bb + nbb)
        if pf_gather:
            issue_gather(j + nb, bb + nb)

    ngrp = n // dd
    for j in range(nbb):
        issue_idx(j, j)
    for j in range(nb):
        issue_gather(j, j)
    for bb in range(dd):
        process(bb, bb, bb >= nb, bb + nbb < n, bb + nb < n)

    def grp(g, _):
        for bb in range(dd):
            process(g * dd + bb, bb, True, True, True)
        return _

    lax.fori_loop(1, ngrp - 1, grp, 0)
    for bb in range(dd):
        j = (ngrp - 1) * dd + bb
        process(j, bb, True, j + nbb < n, j + nb < n)
    for t in range(n - ngrp * dd):
        j = ngrp * dd + t
        process(j, t, True, j + nbb < n, j + nb < n)
    for j in range(n - nb, n):
        wait_scatter(j % nbb)


def _sweep_scratch(nb, ce, width):
    return [
        pltpu.VMEM((2 * nb, ce), jnp.int32),
        pltpu.VMEM((4 * nb, ce), jnp.int32),
        pltpu.VMEM((2 * nb, ce, width), jnp.float32),
    ] + [pltpu.SemaphoreType.DMA] * (6 * nb)


NB1, CE1 = 3, 40
NB2, CE2 = 3, 80


def _sc_pass1(x, edges3, zeros1, zerosc, ones16):
    n_chunks = E // (NC * NS) // CE1

    @functools.partial(
        pl.kernel,
        out_type=[jax.ShapeDtypeStruct((NC, N, D), jnp.float32),
                  jax.ShapeDtypeStruct((NC, N, CW), jnp.float32)],
        mesh=plsc.VectorSubcoreMesh(core_axis_name="c", subcore_axis_name="s"),
        compiler_params=pltpu.CompilerParams(use_tc_tiling_on_sc=False),
        scratch_types=_sweep_scratch(NB1, CE1, D)
        + [pltpu.SemaphoreType.DMA] * NB1
        + [pltpu.VMEM((CE1, CW), jnp.float32),
           pltpu.VMEM_SHARED((N, D), jnp.float32),
           pltpu.VMEM_SHARED((N, CW), jnp.float32)],
    )
    def k(x_h, e3_h, zeros_h, zc_h, ones_h, out_h, outc_h,
          srcb, dstb, rows, *rest):
        sems = list(rest[:7 * NB1])
        onesb, acc, acc_cnt = rest[7 * NB1], rest[7 * NB1 + 1], rest[7 * NB1 + 2]
        c = lax.axis_index("c")
        s = lax.axis_index("s")
        pltpu.sync_copy(ones_h, onesb)
        pltpu.sync_copy(zc_h.at[pl.ds(s * ZR, ZR)], acc_cnt.at[pl.ds(s * ZR, ZR)])

        @pl.when(s == NS - 1)
        def _():
            pltpu.sync_copy(zc_h.at[pl.ds(ZTAIL_BASE + ZR, ZTAIL)],
                            acc_cnt.at[pl.ds(ZTAIL_BASE + ZR, ZTAIL)])

        _zero_and_barrier(zeros_h, acc, s)
        tid = c * NS + s
        edge_base = tid * (E // (NC * NS))
        _emit_sweep(
            n_chunks, NB1, CE1,
            lambda j: e3_h.at[0, pl.ds(edge_base + j * CE1, CE1)],
            lambda j: e3_h.at[2, pl.ds(edge_base + j * CE1, CE1)],
            x_h, acc, srcb, dstb, rows, sems, cnt=(acc_cnt, onesb))
        plsc.subcore_barrier()
        _writeback(acc, out_h, c, s)
        pltpu.sync_copy(acc_cnt.at[pl.ds(s * ZR, ZR)], outc_h.at[c, pl.ds(s * ZR, ZR)])

        @pl.when(s == NS - 1)
        def _():
            pltpu.sync_copy(acc_cnt.at[pl.ds(ZTAIL_BASE + ZR, ZTAIL)],
                            outc_h.at[c, pl.ds(ZTAIL_BASE + ZR, ZTAIL)])

    return k(x, edges3, zeros1, zerosc, ones16)


def _sc_pass2(z_flat, edges3, zeros2):
    n_chunks = E // NS // CE2

    @functools.partial(
        pl.kernel,
        out_type=jax.ShapeDtypeStruct((NC, N, C2), jnp.float32),
        mesh=plsc.VectorSubcoreMesh(core_axis_name="c", subcore_axis_name="s"),
        compiler_params=pltpu.CompilerParams(use_tc_tiling_on_sc=False),
        scratch_types=_sweep_scratch(NB2, CE2, C2) + [pltpu.VMEM_SHARED((N, C2), jnp.float32)],
    )
    def k(z_h, e3_h, zeros_h, out_h, srcb, dstb, rows, *rest):
        sems, acc = list(rest[:6 * NB2]), rest[6 * NB2]
        c = lax.axis_index("c")
        s = lax.axis_index("s")
        edge_base = s * (E // NS)
        _zero_and_barrier(zeros_h, acc, s)
        _emit_sweep(
            n_chunks, NB2, CE2,
            lambda j: e3_h.at[c, pl.ds(edge_base + j * CE2, CE2)],
            lambda j: e3_h.at[2, pl.ds(edge_base + j * CE2, CE2)],
            z_h, acc, srcb, dstb, rows, sems)
        plsc.subcore_barrier()
        _writeback(acc, out_h, c, s)

    return k(z_flat, edges3, zeros2)


RB = 2000

OA, OS, OG = 129, 27, 11
OZ = OA + OS + OG


def _stage_b_body(s1p, cntp, xb, wla, bla, wra, wls, bls, wrs, wlg, blg, wrg,
                  woa, wos, wog, ha, hs, hg, z2, rcnt):
    cnt = cntp[0][:, :1] + cntp[1][:, :1]
    rc = 1.0 / jnp.maximum(cnt, 1.0)
    agg = (s1p[0] + s1p[1]) * rc
    x = xb[...]

    def head(wl, bl, wr):
        return (jnp.dot(agg, wl[...], preferred_element_type=jnp.float32)
                + bl[...]
                + jnp.dot(x, wr[...], preferred_element_type=jnp.float32))

    h_a = head(wla, bla, wra)
    h_s = head(wls, bls, wrs)
    h_g = head(wlg, blg, wrg)
    ha[...] = h_a
    hs[...] = h_s
    hg[...] = h_g
    z = jnp.concatenate(
        [jnp.dot(jnp.maximum(h_a, 0.0), woa[...], preferred_element_type=jnp.float32),
         jnp.dot(jnp.maximum(h_s, 0.0), wos[...], preferred_element_type=jnp.float32),
         jnp.dot(jnp.maximum(h_g, 0.0), wog[...], preferred_element_type=jnp.float32),
         jnp.zeros((RB, 2 * C2 - OZ), jnp.float32)],
        axis=1)
    z2[0] = z[:, :C2]
    z2[1] = z[:, C2:]
    rcnt[...] = jnp.broadcast_to(rc, (RB, 8))


def _tc_stage_b(s1p, cntp, x, wla, bla, wra, wls, bls, wrs, wlg, blg, wrg,
                woa, wos, wog):
    grid = (N // RB,)
    full = lambda shape: pl.BlockSpec(shape, lambda i: (0,) * len(shape))
    row = lambda w: pl.BlockSpec((RB, w), lambda i: (i, 0))
    return pl.pallas_call(
        _stage_b_body,
        grid=grid,
        in_specs=[
            pl.BlockSpec((NC, RB, D), lambda i: (0, i, 0)),
            pl.BlockSpec((NC, RB, CW), lambda i: (0, i, 0)),
            row(D),
            full((D, H)), full((1, H)), full((D, H)),
            full((D, H)), full((1, H)), full((D, H)),
            full((D, H)), full((1, H)), full((D, H)),
            full((H, OA)), full((H, OS)), full((H, OG)),
        ],
        out_specs=[
            row(H), row(H), row(H),
            pl.BlockSpec((NC, RB, C2), lambda i: (0, i, 0)),
            row(8),
        ],
        out_shape=[
            jax.ShapeDtypeStruct((N, H), jnp.float32),
            jax.ShapeDtypeStruct((N, H), jnp.float32),
            jax.ShapeDtypeStruct((N, H), jnp.float32),
            jax.ShapeDtypeStruct((NC, N, C2), jnp.float32),
            jax.ShapeDtypeStruct((N, 8), jnp.float32),
        ],
    )(s1p, cntp, x, wla, bla, wra, wls, bls, wrs, wlg, blg, wrg, woa, wos, wog)


def _log_softmax(xo):
    m = jnp.max(xo, axis=1, keepdims=True)
    e = jnp.exp(xo - m)
    return xo - m - jnp.log(jnp.sum(e, axis=1, keepdims=True))


def _stage_d_body(s2, rcnt, hab, hsb, hgb, ba, ra, bs, rs, bg, rg,
                  ya, ys, yg):
    rc = rcnt[:, :1]
    zagg = jnp.concatenate([s2[0], s2[1]], axis=1) * rc

    def head(lo, hi, hb, bl, wr):
        e = jnp.maximum(hb[...], 0.0)
        return _log_softmax(
            zagg[:, lo:hi] + bl[...]
            + jnp.dot(e, wr[...], preferred_element_type=jnp.float32))

    ya[...] = head(0, OA, hab, ba, ra)
    ys[...] = head(OA, OA + OS, hsb, bs, rs)
    yg[...] = head(OA + OS, OZ, hgb, bg, rg)


def _tc_stage_d(s2, rcnt, ha, hs, hg, ba, ra, bs, rs, bg, rg):
    grid = (N // RB,)
    full = lambda shape: pl.BlockSpec(shape, lambda i: (0,) * len(shape))
    row = lambda w: pl.BlockSpec((RB, w), lambda i: (i, 0))
    return pl.pallas_call(
        _stage_d_body,
        grid=grid,
        in_specs=[
            pl.BlockSpec((NC, RB, C2), lambda i: (0, i, 0)),
            row(8), row(H), row(H), row(H),
            full((1, OA)), full((H, OA)),
            full((1, OS)), full((H, OS)),
            full((1, OG)), full((H, OG)),
        ],
        out_specs=[row(OA), row(OS), row(OG)],
        out_shape=[
            jax.ShapeDtypeStruct((N, OA), jnp.float32),
            jax.ShapeDtypeStruct((N, OS), jnp.float32),
            jax.ShapeDtypeStruct((N, OG), jnp.float32),
        ],
    )(s2, rcnt, ha, hs, hg, ba, ra, bs, rs, bg, rg)


def kernel(x, edge_index,
           Wl1_artist, bl1_artist, Wr1_artist, Wlo_artist, blo_artist, Wro_artist,
           Wl1_style, bl1_style, Wr1_style, Wlo_style, blo_style, Wro_style,
           Wl1_genre, bl1_genre, Wr1_genre, Wlo_genre, blo_genre, Wro_genre):
    src = edge_index[0].astype(jnp.int32)
    dst = edge_index[1].astype(jnp.int32)
    edges3 = jnp.stack([src, src + N, dst])

    zeros1 = jnp.zeros((N, D), jnp.float32)
    zerosc = jnp.zeros((N, CW), jnp.float32)
    zeros2 = jnp.zeros((N, C2), jnp.float32)
    ones16 = jnp.ones((CE1, CW), jnp.float32)

    s1p, cntp = _sc_pass1(x, edges3, zeros1, zerosc, ones16)

    ha, hs, hg, z2, rcnt = _tc_stage_b(
        s1p, cntp, x,
        Wl1_artist, bl1_artist.reshape(1, -1), Wr1_artist,
        Wl1_style, bl1_style.reshape(1, -1), Wr1_style,
        Wl1_genre, bl1_genre.reshape(1, -1), Wr1_genre,
        Wlo_artist, Wlo_style, Wlo_genre)

    s2 = _sc_pass2(z2.reshape(2 * N, C2), edges3, zeros2)

    ya, ys, yg = _tc_stage_d(
        s2, rcnt, ha, hs, hg,
        blo_artist.reshape(1, -1), Wro_artist,
        blo_style.reshape(1, -1), Wro_style,
        blo_genre.reshape(1, -1), Wro_genre)

    return (ha, ya, hs, ys, hg, yg)


# speedup vs baseline: 12.9149x; 1.0056x over previous
import functools

import jax
import jax.numpy as jnp
from jax import lax
from jax.experimental import pallas as pl
from jax.experimental.pallas import tpu as pltpu
from jax.experimental.pallas import tpu_sc as plsc

N = 10000
E = 320000
D = 128
H = 128

NC = 2
NS = 16
CE = 80

CW = 16
C2 = 88

ZR = 624
ZTAIL_BASE = ZR * 15
ZTAIL = N - ZTAIL_BASE - ZR


def _zero_and_barrier(zeros_hbm, acc, s):
    pltpu.sync_copy(zeros_hbm.at[pl.ds(s * ZR, ZR)], acc.at[pl.ds(s * ZR, ZR)])

    @pl.when(s == NS - 1)
    def _():
        pltpu.sync_copy(zeros_hbm.at[pl.ds(ZTAIL_BASE + ZR, ZTAIL)],
                        acc.at[pl.ds(ZTAIL_BASE + ZR, ZTAIL)])

    plsc.subcore_barrier()


def _writeback(acc, out_hbm, c, s):
    pltpu.sync_copy(acc.at[pl.ds(s * ZR, ZR)], out_hbm.at[c, pl.ds(s * ZR, ZR)])

    @pl.when(s == NS - 1)
    def _():
        pltpu.sync_copy(acc.at[pl.ds(ZTAIL_BASE + ZR, ZTAIL)],
                        out_hbm.at[c, pl.ds(ZTAIL_BASE + ZR, ZTAIL)])


def _emit_sweep(n, nb, ce, src_ix, dst_ix, tbl, acc, srcb, dstb, rows, sems,
                cnt=None):
    nbb = 2 * nb
    dd = 2 * nbb
    gsems = sems[:nb]
    ssems = sems[nb:2 * nb]
    isems = sems[2 * nb:2 * nb + nbb]
    dsems = sems[2 * nb + nbb:6 * nb]
    if cnt is not None:
        acc_cnt, ones_ref = cnt
        csems = sems[6 * nb:]

    def issue_idx(j, bb):
        sb, db = bb % nbb, bb % dd
        pltpu.async_copy(src_ix(j), srcb.at[sb], isems[sb])
        pltpu.async_copy(dst_ix(j), dstb.at[db], dsems[sb])

    def issue_gather(j, bb):
        b, sb, rb = bb % nb, bb % nbb, bb % nbb
        pltpu.make_async_copy(src_ix(j), srcb.at[sb], isems[sb]).wait()
        pltpu.async_copy(tbl.at[srcb.at[sb]], rows.at[rb], gsems[b])

    def wait_scatter(bb):
        b, rb = bb % nb, bb % nbb
        pltpu.make_async_copy(rows.at[rb], acc.at[dstb.at[0]], ssems[b]).wait()
        if cnt is not None:
            pltpu.make_async_copy(ones_ref, acc_cnt.at[dstb.at[0]], csems[b]).wait()

    def process(j, bb, wait_prev, pf_idx, pf_gather):
        b, sb, rb, db = bb % nb, bb % nbb, bb % nbb, bb % dd
        if wait_prev:
            wait_scatter(bb + nb)
        pltpu.make_async_copy(tbl.at[srcb.at[sb]], rows.at[rb], gsems[b]).wait()
        pltpu.make_async_copy(dst_ix(j), dstb.at[db], dsems[sb]).wait()
        pltpu.async_copy(rows.at[rb], acc.at[dstb.at[db]], ssems[b], add=True)
        if cnt is not None:
            pltpu.async_copy(ones_ref, acc_cnt.at[dstb.at[db]], csems[b], add=True)
        if pf_idx:
            issue_idx(j + nbb, bb + nbb)
        if pf_gather:
            issue_gather(j + nb, bb + nb)

    ngrp = n // dd
    for j in range(nbb):
        issue_idx(j, j)
    for j in range(nb):
        issue_gather(j, j)
    for bb in range(dd):
        process(bb, bb, bb >= nb, bb + nbb < n, bb + nb < n)

    def grp(g, _):
        for bb in range(dd):
            process(g * dd + bb, bb, True, True, True)
        return _

    lax.fori_loop(1, ngrp - 1, grp, 0)
    for bb in range(dd):
        j = (ngrp - 1) * dd + bb
        process(j, bb, True, j + nbb < n, j + nb < n)
    for t in range(n - ngrp * dd):
        j = ngrp * dd + t
        process(j, t, True, j + nbb < n, j + nb < n)
    for j in range(n - nb, n):
        wait_scatter(j % nbb)


def _sweep_scratch(nb, ce, width):
    return [
        pltpu.VMEM((2 * nb, ce), jnp.int32),
        pltpu.VMEM((4 * nb, ce), jnp.int32),
        pltpu.VMEM((2 * nb, ce, width), jnp.float32),
    ] + [pltpu.SemaphoreType.DMA] * (6 * nb)


NB1, CE1 = 3, 40
NB2, CE2 = 3, 80


def _sc_pass1(x, edges3, zeros1, zerosc, ones16):
    n_chunks = E // (NC * NS) // CE1

    @functools.partial(
        pl.kernel,
        out_type=[jax.ShapeDtypeStruct((NC, N, D), jnp.float32),
                  jax.ShapeDtypeStruct((NC, N, CW), jnp.float32)],
        mesh=plsc.VectorSubcoreMesh(core_axis_name="c", subcore_axis_name="s"),
        compiler_params=pltpu.CompilerParams(use_tc_tiling_on_sc=False),
        scratch_types=_sweep_scratch(NB1, CE1, D)
        + [pltpu.SemaphoreType.DMA] * NB1
        + [pltpu.VMEM((CE1, CW), jnp.float32),
           pltpu.VMEM_SHARED((N, D), jnp.float32),
           pltpu.VMEM_SHARED((N, CW), jnp.float32)],
    )
    def k(x_h, e3_h, zeros_h, zc_h, ones_h, out_h, outc_h,
          srcb, dstb, rows, *rest):
        sems = list(rest[:7 * NB1])
        onesb, acc, acc_cnt = rest[7 * NB1], rest[7 * NB1 + 1], rest[7 * NB1 + 2]
        c = lax.axis_index("c")
        s = lax.axis_index("s")
        pltpu.sync_copy(ones_h, onesb)
        pltpu.sync_copy(zc_h.at[pl.ds(s * ZR, ZR)], acc_cnt.at[pl.ds(s * ZR, ZR)])

        @pl.when(s == NS - 1)
        def _():
            pltpu.sync_copy(zc_h.at[pl.ds(ZTAIL_BASE + ZR, ZTAIL)],
                            acc_cnt.at[pl.ds(ZTAIL_BASE + ZR, ZTAIL)])

        _zero_and_barrier(zeros_h, acc, s)
        tid = c * NS + s
        edge_base = tid * (E // (NC * NS))
        _emit_sweep(
            n_chunks, NB1, CE1,
            lambda j: e3_h.at[0, pl.ds(edge_base + j * CE1, CE1)],
            lambda j: e3_h.at[2, pl.ds(edge_base + j * CE1, CE1)],
            x_h, acc, srcb, dstb, rows, sems, cnt=(acc_cnt, onesb))
        plsc.subcore_barrier()
        _writeback(acc, out_h, c, s)
        pltpu.sync_copy(acc_cnt.at[pl.ds(s * ZR, ZR)], outc_h.at[c, pl.ds(s * ZR, ZR)])

        @pl.when(s == NS - 1)
        def _():
            pltpu.sync_copy(acc_cnt.at[pl.ds(ZTAIL_BASE + ZR, ZTAIL)],
                            outc_h.at[c, pl.ds(ZTAIL_BASE + ZR, ZTAIL)])

    return k(x, edges3, zeros1, zerosc, ones16)


def _sc_pass2(z_flat, edges3, zeros2):
    n_chunks = E // NS // CE2

    @functools.partial(
        pl.kernel,
        out_type=jax.ShapeDtypeStruct((NC, N, C2), jnp.float32),
        mesh=plsc.VectorSubcoreMesh(core_axis_name="c", subcore_axis_name="s"),
        compiler_params=pltpu.CompilerParams(use_tc_tiling_on_sc=False),
        scratch_types=_sweep_scratch(NB2, CE2, C2) + [pltpu.VMEM_SHARED((N, C2), jnp.float32)],
    )
    def k(z_h, e3_h, zeros_h, out_h, srcb, dstb, rows, *rest):
        sems, acc = list(rest[:6 * NB2]), rest[6 * NB2]
        c = lax.axis_index("c")
        s = lax.axis_index("s")
        edge_base = s * (E // NS)
        _zero_and_barrier(zeros_h, acc, s)
        _emit_sweep(
            n_chunks, NB2, CE2,
            lambda j: e3_h.at[c, pl.ds(edge_base + j * CE2, CE2)],
            lambda j: e3_h.at[2, pl.ds(edge_base + j * CE2, CE2)],
            z_h, acc, srcb, dstb, rows, sems)
        plsc.subcore_barrier()
        _writeback(acc, out_h, c, s)

    return k(z_flat, edges3, zeros2)


RB = 2000

OA, OS, OG = 129, 27, 11
OZ = OA + OS + OG


def _stage_b_body(s1p, cntp, xb, wla, bla, wra, wls, bls, wrs, wlg, blg, wrg,
                  woa, wos, wog, ba, ra, bs, rs, bg, rg, ha, hs, hg, z2, w_out, rcnt):
    cnt = cntp[0][:, :1] + cntp[1][:, :1]
    rc = 1.0 / jnp.maximum(cnt, 1.0)
    agg = (s1p[0] + s1p[1]) * rc
    x = xb[...]

    def head(wl, bl, wr):
        return (jnp.dot(agg, wl[...], preferred_element_type=jnp.float32)
                + bl[...]
                + jnp.dot(x, wr[...], preferred_element_type=jnp.float32))

    h_a = head(wla, bla, wra)
    h_s = head(wls, bls, wrs)
    h_g = head(wlg, blg, wrg)
    ha[...] = h_a
    hs[...] = h_s
    hg[...] = h_g
    e_a = jnp.maximum(h_a, 0.0)
    e_s = jnp.maximum(h_s, 0.0)
    e_g = jnp.maximum(h_g, 0.0)
    pad = jnp.zeros((RB, 2 * C2 - OZ), jnp.float32)
    z = jnp.concatenate(
        [jnp.dot(e_a, woa[...], preferred_element_type=jnp.float32),
         jnp.dot(e_s, wos[...], preferred_element_type=jnp.float32),
         jnp.dot(e_g, wog[...], preferred_element_type=jnp.float32),
         pad],
        axis=1)
    z2[0] = z[:, :C2]
    z2[1] = z[:, C2:]
    w_out[...] = jnp.concatenate(
        [jnp.dot(e_a, ra[...], preferred_element_type=jnp.float32) + ba[...],
         jnp.dot(e_s, rs[...], preferred_element_type=jnp.float32) + bs[...],
         jnp.dot(e_g, rg[...], preferred_element_type=jnp.float32) + bg[...],
         pad],
        axis=1)
    rcnt[...] = jnp.broadcast_to(rc, (RB, 8))


def _tc_stage_b(s1p, cntp, x, wla, bla, wra, wls, bls, wrs, wlg, blg, wrg,
                woa, wos, wog, ba, ra, bs, rs, bg, rg):
    grid = (N // RB,)
    full = lambda shape: pl.BlockSpec(shape, lambda i: (0,) * len(shape))
    row = lambda w: pl.BlockSpec((RB, w), lambda i: (i, 0))
    return pl.pallas_call(
        _stage_b_body,
        grid=grid,
        in_specs=[
            pl.BlockSpec((NC, RB, D), lambda i: (0, i, 0)),
            pl.BlockSpec((NC, RB, CW), lambda i: (0, i, 0)),
            row(D),
            full((D, H)), full((1, H)), full((D, H)),
            full((D, H)), full((1, H)), full((D, H)),
            full((D, H)), full((1, H)), full((D, H)),
            full((H, OA)), full((H, OS)), full((H, OG)),
            full((1, OA)), full((H, OA)),
            full((1, OS)), full((H, OS)),
            full((1, OG)), full((H, OG)),
        ],
        out_specs=[
            row(H), row(H), row(H),
            pl.BlockSpec((NC, RB, C2), lambda i: (0, i, 0)),
            row(2 * C2),
            row(8),
        ],
        out_shape=[
            jax.ShapeDtypeStruct((N, H), jnp.float32),
            jax.ShapeDtypeStruct((N, H), jnp.float32),
            jax.ShapeDtypeStruct((N, H), jnp.float32),
            jax.ShapeDtypeStruct((NC, N, C2), jnp.float32),
            jax.ShapeDtypeStruct((N, 2 * C2), jnp.float32),
            jax.ShapeDtypeStruct((N, 8), jnp.float32),
        ],
    )(s1p, cntp, x, wla, bla, wra, wls, bls, wrs, wlg, blg, wrg, woa, wos, wog,
      ba, ra, bs, rs, bg, rg)


def _log_softmax(xo):
    m = jnp.max(xo, axis=1, keepdims=True)
    e = jnp.exp(xo - m)
    return xo - m - jnp.log(jnp.sum(e, axis=1, keepdims=True))


def _stage_d_body(s2, rcnt, wb, ya, ys, yg):
    rc = rcnt[:, :1]
    xo = jnp.concatenate([s2[0], s2[1]], axis=1) * rc + wb[...]
    ya[...] = _log_softmax(xo[:, :OA])
    ys[...] = _log_softmax(xo[:, OA:OA + OS])
    yg[...] = _log_softmax(xo[:, OA + OS:OZ])


def _tc_stage_d(s2, rcnt, w):
    grid = (N // RB,)
    row = lambda wd: pl.BlockSpec((RB, wd), lambda i: (i, 0))
    return pl.pallas_call(
        _stage_d_body,
        grid=grid,
        in_specs=[
            pl.BlockSpec((NC, RB, C2), lambda i: (0, i, 0)),
            row(8), row(2 * C2),
        ],
        out_specs=[row(OA), row(OS), row(OG)],
        out_shape=[
            jax.ShapeDtypeStruct((N, OA), jnp.float32),
            jax.ShapeDtypeStruct((N, OS), jnp.float32),
            jax.ShapeDtypeStruct((N, OG), jnp.float32),
        ],
    )(s2, rcnt, w)


def kernel(x, edge_index,
           Wl1_artist, bl1_artist, Wr1_artist, Wlo_artist, blo_artist, Wro_artist,
           Wl1_style, bl1_style, Wr1_style, Wlo_style, blo_style, Wro_style,
           Wl1_genre, bl1_genre, Wr1_genre, Wlo_genre, blo_genre, Wro_genre):
    src = edge_index[0].astype(jnp.int32)
    dst = edge_index[1].astype(jnp.int32)
    edges3 = jnp.stack([src, src + N, dst])

    zeros1 = jnp.zeros((N, D), jnp.float32)
    zerosc = jnp.zeros((N, CW), jnp.float32)
    zeros2 = jnp.zeros((N, C2), jnp.float32)
    ones16 = jnp.ones((CE1, CW), jnp.float32)

    s1p, cntp = _sc_pass1(x, edges3, zeros1, zerosc, ones16)

    ha, hs, hg, z2, w, rcnt = _tc_stage_b(
        s1p, cntp, x,
        Wl1_artist, bl1_artist.reshape(1, -1), Wr1_artist,
        Wl1_style, bl1_style.reshape(1, -1), Wr1_style,
        Wl1_genre, bl1_genre.reshape(1, -1), Wr1_genre,
        Wlo_artist, Wlo_style, Wlo_genre,
        blo_artist.reshape(1, -1), Wro_artist,
        blo_style.reshape(1, -1), Wro_style,
        blo_genre.reshape(1, -1), Wro_genre)

    s2 = _sc_pass2(z2.reshape(2 * N, C2), edges3, zeros2)

    ya, ys, yg = _tc_stage_d(s2, rcnt, w)

    return (ha, ya, hs, ys, hg, yg)


# speedup vs baseline: 13.0174x; 1.0079x over previous
import functools

import jax
import jax.numpy as jnp
from jax import lax
from jax.experimental import pallas as pl
from jax.experimental.pallas import tpu as pltpu
from jax.experimental.pallas import tpu_sc as plsc

N = 10000
E = 320000
D = 128
H = 128

NC = 2
NS = 16
CE = 80

CW = 16
C2 = 88

ZR = 624
ZTAIL_BASE = ZR * 15
ZTAIL = N - ZTAIL_BASE - ZR


def _zero_and_barrier(zeros_hbm, acc, s):
    pltpu.sync_copy(zeros_hbm.at[pl.ds(s * ZR, ZR)], acc.at[pl.ds(s * ZR, ZR)])

    @pl.when(s == NS - 1)
    def _():
        pltpu.sync_copy(zeros_hbm.at[pl.ds(ZTAIL_BASE + ZR, ZTAIL)],
                        acc.at[pl.ds(ZTAIL_BASE + ZR, ZTAIL)])

    plsc.subcore_barrier()


def _writeback(acc, out_hbm, c, s):
    pltpu.sync_copy(acc.at[pl.ds(s * ZR, ZR)], out_hbm.at[c, pl.ds(s * ZR, ZR)])

    @pl.when(s == NS - 1)
    def _():
        pltpu.sync_copy(acc.at[pl.ds(ZTAIL_BASE + ZR, ZTAIL)],
                        out_hbm.at[c, pl.ds(ZTAIL_BASE + ZR, ZTAIL)])


def _emit_sweep(n, nb, ce, src_ix, dst_ix, tbl, acc, srcb, dstb, rows, sems,
                cnt=None):
    nbb = 2 * nb
    dd = 2 * nbb
    gsems = sems[:nb]
    ssems = sems[nb:2 * nb]
    isems = sems[2 * nb:2 * nb + nbb]
    dsems = sems[2 * nb + nbb:6 * nb]
    if cnt is not None:
        acc_cnt, ones_ref = cnt
        csems = sems[6 * nb:]

    def issue_idx(j, bb):
        sb, db = bb % nbb, bb % dd
        pltpu.async_copy(src_ix(j), srcb.at[sb], isems[sb])
        pltpu.async_copy(dst_ix(j), dstb.at[db], dsems[sb])

    def issue_gather(j, bb):
        b, sb, rb = bb % nb, bb % nbb, bb % nbb
        pltpu.make_async_copy(src_ix(j), srcb.at[sb], isems[sb]).wait()
        pltpu.async_copy(tbl.at[srcb.at[sb]], rows.at[rb], gsems[b])

    def wait_scatter(bb):
        b, rb = bb % nb, bb % nbb
        pltpu.make_async_copy(rows.at[rb], acc.at[dstb.at[0]], ssems[b]).wait()
        if cnt is not None:
            pltpu.make_async_copy(ones_ref, acc_cnt.at[dstb.at[0]], csems[b]).wait()

    def process(j, bb, wait_prev, pf_idx, pf_gather):
        b, sb, rb, db = bb % nb, bb % nbb, bb % nbb, bb % dd
        if wait_prev:
            wait_scatter(bb + nb)
        pltpu.make_async_copy(tbl.at[srcb.at[sb]], rows.at[rb], gsems[b]).wait()
        pltpu.make_async_copy(dst_ix(j), dstb.at[db], dsems[sb]).wait()
        pltpu.async_copy(rows.at[rb], acc.at[dstb.at[db]], ssems[b], add=True)
        if cnt is not None:
            pltpu.async_copy(ones_ref, acc_cnt.at[dstb.at[db]], csems[b], add=True)
        if pf_idx:
            issue_idx(j + nbb, bb + nbb)
        if pf_gather:
            issue_gather(j + nb, bb + nb)

    ngrp = n // dd
    for j in range(nbb):
        issue_idx(j, j)
    for j in range(nb):
        issue_gather(j, j)
    for bb in range(dd):
        process(bb, bb, bb >= nb, bb + nbb < n, bb + nb < n)

    def grp(g, _):
        for bb in range(dd):
            process(g * dd + bb, bb, True, True, True)
        return _

    lax.fori_loop(1, ngrp - 1, grp, 0)
    for bb in range(dd):
        j = (ngrp - 1) * dd + bb
        process(j, bb, True, j + nbb < n, j + nb < n)
    for t in range(n - ngrp * dd):
        j = ngrp * dd + t
        process(j, t, True, j + nbb < n, j + nb < n)
    for j in range(n - nb, n):
        wait_scatter(j % nbb)


def _sweep_scratch(nb, ce, width):
    return [
        pltpu.VMEM((2 * nb, ce), jnp.int32),
        pltpu.VMEM((4 * nb, ce), jnp.int32),
        pltpu.VMEM((2 * nb, ce, width), jnp.float32),
    ] + [pltpu.SemaphoreType.DMA] * (6 * nb)


NB1, CE1 = 3, 40
NB2, CE2 = 4, 80


def _sc_pass1(x, edges3, zeros1, zerosc, ones16):
    n_chunks = E // (NC * NS) // CE1

    @functools.partial(
        pl.kernel,
        out_type=[jax.ShapeDtypeStruct((NC, N, D), jnp.float32),
                  jax.ShapeDtypeStruct((NC, N, CW), jnp.float32)],
        mesh=plsc.VectorSubcoreMesh(core_axis_name="c", subcore_axis_name="s"),
        compiler_params=pltpu.CompilerParams(use_tc_tiling_on_sc=False),
        scratch_types=_sweep_scratch(NB1, CE1, D)
        + [pltpu.SemaphoreType.DMA] * NB1
        + [pltpu.VMEM((CE1, CW), jnp.float32),
           pltpu.VMEM_SHARED((N, D), jnp.float32),
           pltpu.VMEM_SHARED((N, CW), jnp.float32)],
    )
    def k(x_h, e3_h, zeros_h, zc_h, ones_h, out_h, outc_h,
          srcb, dstb, rows, *rest):
        sems = list(rest[:7 * NB1])
        onesb, acc, acc_cnt = rest[7 * NB1], rest[7 * NB1 + 1], rest[7 * NB1 + 2]
        c = lax.axis_index("c")
        s = lax.axis_index("s")
        pltpu.sync_copy(ones_h, onesb)
        pltpu.sync_copy(zc_h.at[pl.ds(s * ZR, ZR)], acc_cnt.at[pl.ds(s * ZR, ZR)])

        @pl.when(s == NS - 1)
        def _():
            pltpu.sync_copy(zc_h.at[pl.ds(ZTAIL_BASE + ZR, ZTAIL)],
                            acc_cnt.at[pl.ds(ZTAIL_BASE + ZR, ZTAIL)])

        _zero_and_barrier(zeros_h, acc, s)
        tid = c * NS + s
        edge_base = tid * (E // (NC * NS))
        _emit_sweep(
            n_chunks, NB1, CE1,
            lambda j: e3_h.at[0, pl.ds(edge_base + j * CE1, CE1)],
            lambda j: e3_h.at[2, pl.ds(edge_base + j * CE1, CE1)],
            x_h, acc, srcb, dstb, rows, sems, cnt=(acc_cnt, onesb))
        plsc.subcore_barrier()
        _writeback(acc, out_h, c, s)
        pltpu.sync_copy(acc_cnt.at[pl.ds(s * ZR, ZR)], outc_h.at[c, pl.ds(s * ZR, ZR)])

        @pl.when(s == NS - 1)
        def _():
            pltpu.sync_copy(acc_cnt.at[pl.ds(ZTAIL_BASE + ZR, ZTAIL)],
                            outc_h.at[c, pl.ds(ZTAIL_BASE + ZR, ZTAIL)])

    return k(x, edges3, zeros1, zerosc, ones16)


def _sc_pass2(z_flat, edges3, zeros2):
    n_chunks = E // NS // CE2

    @functools.partial(
        pl.kernel,
        out_type=jax.ShapeDtypeStruct((NC, N, C2), jnp.float32),
        mesh=plsc.VectorSubcoreMesh(core_axis_name="c", subcore_axis_name="s"),
        compiler_params=pltpu.CompilerParams(use_tc_tiling_on_sc=False),
        scratch_types=_sweep_scratch(NB2, CE2, C2) + [pltpu.VMEM_SHARED((N, C2), jnp.float32)],
    )
    def k(z_h, e3_h, zeros_h, out_h, srcb, dstb, rows, *rest):
        sems, acc = list(rest[:6 * NB2]), rest[6 * NB2]
        c = lax.axis_index("c")
        s = lax.axis_index("s")
        edge_base = s * (E // NS)
        _zero_and_barrier(zeros_h, acc, s)
        _emit_sweep(
            n_chunks, NB2, CE2,
            lambda j: e3_h.at[c, pl.ds(edge_base + j * CE2, CE2)],
            lambda j: e3_h.at[2, pl.ds(edge_base + j * CE2, CE2)],
            z_h, acc, srcb, dstb, rows, sems)
        plsc.subcore_barrier()
        _writeback(acc, out_h, c, s)

    return k(z_flat, edges3, zeros2)


RB = 2000

OA, OS, OG = 129, 27, 11
OZ = OA + OS + OG


def _stage_b_body(s1p, cntp, xb, wla, bla, wra, wls, bls, wrs, wlg, blg, wrg,
                  woa, wos, wog, ba, ra, bs, rs, bg, rg, ha, hs, hg, z2, w_out, rcnt):
    cnt = cntp[0][:, :1] + cntp[1][:, :1]
    rc = 1.0 / jnp.maximum(cnt, 1.0)
    agg = (s1p[0] + s1p[1]) * rc
    x = xb[...]

    def head(wl, bl, wr):
        return (jnp.dot(agg, wl[...], preferred_element_type=jnp.float32)
                + bl[...]
                + jnp.dot(x, wr[...], preferred_element_type=jnp.float32))

    h_a = head(wla, bla, wra)
    h_s = head(wls, bls, wrs)
    h_g = head(wlg, blg, wrg)
    ha[...] = h_a
    hs[...] = h_s
    hg[...] = h_g
    e_a = jnp.maximum(h_a, 0.0)
    e_s = jnp.maximum(h_s, 0.0)
    e_g = jnp.maximum(h_g, 0.0)
    pad = jnp.zeros((RB, 2 * C2 - OZ), jnp.float32)
    z = jnp.concatenate(
        [jnp.dot(e_a, woa[...], preferred_element_type=jnp.float32),
         jnp.dot(e_s, wos[...], preferred_element_type=jnp.float32),
         jnp.dot(e_g, wog[...], preferred_element_type=jnp.float32),
         pad],
        axis=1)
    z2[0] = z[:, :C2]
    z2[1] = z[:, C2:]
    w_out[...] = jnp.concatenate(
        [jnp.dot(e_a, ra[...], preferred_element_type=jnp.float32) + ba[...],
         jnp.dot(e_s, rs[...], preferred_element_type=jnp.float32) + bs[...],
         jnp.dot(e_g, rg[...], preferred_element_type=jnp.float32) + bg[...],
         pad],
        axis=1)
    rcnt[...] = jnp.broadcast_to(rc, (RB, 8))


def _tc_stage_b(s1p, cntp, x, wla, bla, wra, wls, bls, wrs, wlg, blg, wrg,
                woa, wos, wog, ba, ra, bs, rs, bg, rg):
    grid = (N // RB,)
    full = lambda shape: pl.BlockSpec(shape, lambda i: (0,) * len(shape))
    row = lambda w: pl.BlockSpec((RB, w), lambda i: (i, 0))
    return pl.pallas_call(
        _stage_b_body,
        grid=grid,
        in_specs=[
            pl.BlockSpec((NC, RB, D), lambda i: (0, i, 0)),
            pl.BlockSpec((NC, RB, CW), lambda i: (0, i, 0)),
            row(D),
            full((D, H)), full((1, H)), full((D, H)),
            full((D, H)), full((1, H)), full((D, H)),
            full((D, H)), full((1, H)), full((D, H)),
            full((H, OA)), full((H, OS)), full((H, OG)),
            full((1, OA)), full((H, OA)),
            full((1, OS)), full((H, OS)),
            full((1, OG)), full((H, OG)),
        ],
        out_specs=[
            row(H), row(H), row(H),
            pl.BlockSpec((NC, RB, C2), lambda i: (0, i, 0)),
            row(2 * C2),
            row(8),
        ],
        out_shape=[
            jax.ShapeDtypeStruct((N, H), jnp.float32),
            jax.ShapeDtypeStruct((N, H), jnp.float32),
            jax.ShapeDtypeStruct((N, H), jnp.float32),
            jax.ShapeDtypeStruct((NC, N, C2), jnp.float32),
            jax.ShapeDtypeStruct((N, 2 * C2), jnp.float32),
            jax.ShapeDtypeStruct((N, 8), jnp.float32),
        ],
    )(s1p, cntp, x, wla, bla, wra, wls, bls, wrs, wlg, blg, wrg, woa, wos, wog,
      ba, ra, bs, rs, bg, rg)


def _log_softmax(xo):
    m = jnp.max(xo, axis=1, keepdims=True)
    e = jnp.exp(xo - m)
    return xo - m - jnp.log(jnp.sum(e, axis=1, keepdims=True))


def _stage_d_body(s2, rcnt, wb, ya, ys, yg):
    rc = rcnt[:, :1]
    xo = jnp.concatenate([s2[0], s2[1]], axis=1) * rc + wb[...]
    ya[...] = _log_softmax(xo[:, :OA])
    ys[...] = _log_softmax(xo[:, OA:OA + OS])
    yg[...] = _log_softmax(xo[:, OA + OS:OZ])


def _tc_stage_d(s2, rcnt, w):
    grid = (N // RB,)
    row = lambda wd: pl.BlockSpec((RB, wd), lambda i: (i, 0))
    return pl.pallas_call(
        _stage_d_body,
        grid=grid,
        in_specs=[
            pl.BlockSpec((NC, RB, C2), lambda i: (0, i, 0)),
            row(8), row(2 * C2),
        ],
        out_specs=[row(OA), row(OS), row(OG)],
        out_shape=[
            jax.ShapeDtypeStruct((N, OA), jnp.float32),
            jax.ShapeDtypeStruct((N, OS), jnp.float32),
            jax.ShapeDtypeStruct((N, OG), jnp.float32),
        ],
    )(s2, rcnt, w)


def kernel(x, edge_index,
           Wl1_artist, bl1_artist, Wr1_artist, Wlo_artist, blo_artist, Wro_artist,
           Wl1_style, bl1_style, Wr1_style, Wlo_style, blo_style, Wro_style,
           Wl1_genre, bl1_genre, Wr1_genre, Wlo_genre, blo_genre, Wro_genre):
    src = edge_index[0].astype(jnp.int32)
    dst = edge_index[1].astype(jnp.int32)
    edges3 = jnp.stack([src, src + N, dst])

    zeros1 = jnp.zeros((N, D), jnp.float32)
    zerosc = jnp.zeros((N, CW), jnp.float32)
    zeros2 = jnp.zeros((N, C2), jnp.float32)
    ones16 = jnp.ones((CE1, CW), jnp.float32)

    s1p, cntp = _sc_pass1(x, edges3, zeros1, zerosc, ones16)

    ha, hs, hg, z2, w, rcnt = _tc_stage_b(
        s1p, cntp, x,
        Wl1_artist, bl1_artist.reshape(1, -1), Wr1_artist,
        Wl1_style, bl1_style.reshape(1, -1), Wr1_style,
        Wl1_genre, bl1_genre.reshape(1, -1), Wr1_genre,
        Wlo_artist, Wlo_style, Wlo_genre,
        blo_artist.reshape(1, -1), Wro_artist,
        blo_style.reshape(1, -1), Wro_style,
        blo_genre.reshape(1, -1), Wro_genre)

    s2 = _sc_pass2(z2.reshape(2 * N, C2), edges3, zeros2)

    ya, ys, yg = _tc_stage_d(s2, rcnt, w)

    return (ha, ya, hs, ys, hg, yg)
